# baseline (speedup 1.0000x reference)
import functools

import jax
import jax.numpy as jnp
from jax import lax
from jax.experimental import pallas as pl
from jax.experimental.pallas import tpu as pltpu
from jax.experimental.pallas import tpu_sc as plsc

N = 50000
D = 128
E = 300000
R = 2
NL = 2

NC, NS, LANES = 2, 16, 16
NW = NC * NS

NCHUNK = 8
CHUNK = 6272
NPAD = NCHUNK * CHUNK
CPSC = NCHUNK // NC
DUMP = CHUNK
RPT = CHUNK // NS

EPW = 9376
EPAD = EPW * NW
BATCH = 128
NBAT = 80
CAP = NBAT * BATCH
NPIPE = 3

TBLK = 512

f32 = jnp.float32
i32 = jnp.int32


@functools.lru_cache(maxsize=1)
def _sc_kernels():
    mesh = plsc.VectorSubcoreMesh(core_axis_name="c", subcore_axis_name="s",
                                  num_cores=NC, num_subcores=NS)
    cparams = pltpu.CompilerParams(needs_layout_passes=False)
    bin_k = functools.partial(
        pl.kernel,
        out_type=[
            jax.ShapeDtypeStruct((R * NCHUNK * NW * CAP,), i32),
            jax.ShapeDtypeStruct((R * NCHUNK * NW * CAP,), i32),
            jax.ShapeDtypeStruct((R * NCHUNK * NW * CAP,), i32),
            jax.ShapeDtypeStruct((R * NW * NCHUNK * LANES,), i32),
        ],
        mesh=mesh,
        compiler_params=cparams,
        scratch_types=[
            pltpu.VMEM((EPW,), i32),
            pltpu.VMEM((EPW,), i32),
            pltpu.VMEM((CAP,), i32),
            pltpu.VMEM((CAP,), i32),
            pltpu.VMEM((CAP,), i32),
            pltpu.VMEM((NCHUNK * LANES,), i32),
        ],
    )(_bin_body)
    s_k = functools.partial(
        pl.kernel,
        out_type=jax.ShapeDtypeStruct((NPAD, D), f32),
        mesh=mesh,
        compiler_params=cparams,
        scratch_types=[
            pltpu.VMEM((NBAT, BATCH), i32),
            pltpu.VMEM((NBAT, BATCH), i32),
            pltpu.VMEM((NPIPE, BATCH, D), f32),
            pltpu.VMEM((NC * NCHUNK * LANES,), i32),
            pltpu.VMEM_SHARED((CHUNK + 1, D), f32),
            [pltpu.SemaphoreType.DMA] * NPIPE,
        ],
    )(_s_body)
    agg_k = functools.partial(
        pl.kernel,
        out_type=jax.ShapeDtypeStruct((NPAD, D), f32),
        mesh=mesh,
        compiler_params=cparams,
        scratch_types=[
            pltpu.VMEM((NBAT, BATCH), i32),
            pltpu.VMEM((NBAT, BATCH), i32),
            pltpu.VMEM((NPIPE, BATCH, D), f32),
            pltpu.VMEM((R * NC * NCHUNK * LANES,), i32),
            pltpu.VMEM_SHARED((CHUNK + 1, D), f32),
            [pltpu.SemaphoreType.DMA] * NPIPE,
        ],
    )(_agg_body)
    return bin_k, s_k, agg_k


def _bin_body(src_hbm, dst_hbm, bin_src, bin_dloc, bin_eid, counts,
              st_s, st_d, seg_s, seg_dl, seg_e, cnt_st):
    cid_ax = lax.axis_index("c")
    sid_ax = lax.axis_index("s")
    wid = sid_ax * NC + cid_ax
    lanes = lax.iota(i32, LANES)
    zeros16 = jnp.zeros((LANES,), i32)
    dump16 = jnp.full((LANES,), DUMP, i32)
    epad16 = jnp.full((LANES,), E, i32)
    ebase = wid * EPW

    for r in range(R):
        pltpu.sync_copy(src_hbm.at[pl.ds(r * EPAD + ebase, EPW)], st_s)
        pltpu.sync_copy(dst_hbm.at[pl.ds(r * EPAD + ebase, EPW)], st_d)
        for c in range(NCHUNK):
            lo, hi = c * CHUNK, (c + 1) * CHUNK

            def body(i, off):
                s = st_s[pl.ds(i * LANES, LANES)]
                d = st_d[pl.ds(i * LANES, LANES)]
                eid = (ebase + i * LANES) + lanes
                m = (d >= lo) & (d < hi)
                plsc.store_compressed(seg_s.at[pl.ds(off, LANES)], s,
                                      mask=m)
                plsc.store_compressed(seg_dl.at[pl.ds(off, LANES)], d - lo,
                                      mask=m)
                plsc.store_compressed(seg_e.at[pl.ds(off, LANES)], eid,
                                      mask=m)
                return off + plsc.all_reduce_population_count(m)[0]

            off = lax.fori_loop(0, EPW // LANES, body, jnp.int32(0))
            for k in range(NPIPE * BATCH // LANES):
                seg_s[pl.ds(off + k * LANES, LANES)] = zeros16
                seg_dl[pl.ds(off + k * LANES, LANES)] = dump16
                seg_e[pl.ds(off + k * LANES, LANES)] = epad16
            cnt_st[pl.ds(c * LANES, LANES)] = jnp.full((LANES,), off, i32)
            row = (r * NCHUNK + c) * NW + wid
            pltpu.sync_copy(seg_s, bin_src.at[pl.ds(row * CAP, CAP)])
            pltpu.sync_copy(seg_dl, bin_dloc.at[pl.ds(row * CAP, CAP)])
            pltpu.sync_copy(seg_e, bin_eid.at[pl.ds(row * CAP, CAP)])
        pltpu.sync_copy(
            cnt_st,
            counts.at[pl.ds((r * NW + wid) * NCHUNK * LANES,
                            NCHUNK * LANES)])


def _s_body(a_hbm, bin_eid, bin_dloc, counts, zrows,
            s_out, eidv, dlocv, arows, cnt, acc, sems):
    cid_ax = lax.axis_index("c")
    sid_ax = lax.axis_index("s")
    pltpu.sync_copy(
        counts.at[pl.ds(2 * sid_ax * NCHUNK * LANES, 2 * NCHUNK * LANES)],
        cnt)

    for cc in range(CPSC):
        chunk = cid_ax + NC * cc
        pltpu.sync_copy(zrows, acc.at[pl.ds(sid_ax * RPT, RPT)])
        plsc.subcore_barrier()
        for sl in range(2):
            seg = 2 * sid_ax + sl
            n = cnt[pl.ds((sl * NCHUNK + chunk) * LANES, LANES)][0]
            row = chunk * NW + seg
            pltpu.sync_copy(bin_eid.at[pl.ds(row * NBAT, NBAT)], eidv)
            pltpu.sync_copy(bin_dloc.at[pl.ds(row * NBAT, NBAT)], dlocv)
            ng = (n + NPIPE * BATCH - 1) // (NPIPE * BATCH)

            def bbody(g, _):
                b0 = g * NPIPE
                cps = [pltpu.async_copy(a_hbm.at[eidv.at[b0 + k]],
                                        arows.at[k], sems[k])
                       for k in range(NPIPE)]
                for k in range(NPIPE):
                    cps[k].wait()
                    pltpu.sync_copy(arows.at[k], acc.at[dlocv.at[b0 + k]],
                                    add=True)
                return 0

            lax.fori_loop(0, ng, bbody, 0)
        plsc.subcore_barrier()
        pltpu.sync_copy(
            acc.at[pl.ds(sid_ax * RPT, RPT)],
            s_out.at[pl.ds(chunk * CHUNK + sid_ax * RPT, RPT)])
        plsc.subcore_barrier()


def _agg_body(m0_hbm, m1_hbm, base_hbm, bin_src, bin_dloc, counts,
              agg, srcv, dlocv, rows, cnt, acc, sems):
    cid_ax = lax.axis_index("c")
    sid_ax = lax.axis_index("s")
    for r in range(R):
        pltpu.sync_copy(
            counts.at[pl.ds((r * NW + 2 * sid_ax) * NCHUNK * LANES,
                            2 * NCHUNK * LANES)],
            cnt.at[pl.ds(r * 2 * NCHUNK * LANES, 2 * NCHUNK * LANES)])

    for cc in range(CPSC):
        chunk = cid_ax + NC * cc
        rowbase = chunk * CHUNK + sid_ax * RPT
        pltpu.sync_copy(base_hbm.at[pl.ds(rowbase, RPT)],
                        acc.at[pl.ds(sid_ax * RPT, RPT)])
        plsc.subcore_barrier()
        for r in range(R):
            m_hbm = m0_hbm if r == 0 else m1_hbm
            for sl in range(2):
                seg = 2 * sid_ax + sl
                n = cnt[pl.ds(((r * 2 + sl) * NCHUNK + chunk) * LANES,
                              LANES)][0]
                row = (r * NCHUNK + chunk) * NW + seg
                pltpu.sync_copy(bin_src.at[pl.ds(row * NBAT, NBAT)], srcv)
                pltpu.sync_copy(bin_dloc.at[pl.ds(row * NBAT, NBAT)],
                                dlocv)
                ng = (n + NPIPE * BATCH - 1) // (NPIPE * BATCH)

                def gbody(g, _, m_hbm=m_hbm):
                    b0 = g * NPIPE
                    cps = [pltpu.async_copy(m_hbm.at[srcv.at[b0 + k]],
                                            rows.at[k], sems[k])
                           for k in range(NPIPE)]
                    for k in range(NPIPE):
                        cps[k].wait()
                        pltpu.sync_copy(rows.at[k],
                                        acc.at[dlocv.at[b0 + k]], add=True)
                    return 0

                lax.fori_loop(0, ng, gbody, 0)
        plsc.subcore_barrier()
        pltpu.sync_copy(acc.at[pl.ds(sid_ax * RPT, RPT)],
                        agg.at[pl.ds(rowbase, RPT)])
        plsc.subcore_barrier()


def _ln(x, g, b):
    m = jnp.mean(x, axis=-1, keepdims=True)
    xc = x - m
    v = jnp.mean(xc * xc, axis=-1, keepdims=True)
    return g * xc * lax.rsqrt(v + 1e-5) + b


def _mats_body(h, s, w0, b0, w1, b1, ew, m0_ref, m1_ref, base_ref):
    m0_ref[...] = jnp.dot(h, w0, preferred_element_type=f32) + b0
    m1_ref[...] = jnp.dot(h, w1, preferred_element_type=f32) + b1
    base_ref[...] = jnp.dot(s, ew, preferred_element_type=f32)


def _t_in_body(nf_ref, iw_ref, ib_ref, w0_ref, b0_ref, w1_ref, b1_ref,
               s_ref, ew_ref, h_ref, m0_ref, m1_ref, base_ref):
    h = jnp.dot(nf_ref[...], iw_ref[...], preferred_element_type=f32) \
        + ib_ref[...]
    h_ref[...] = h
    _mats_body(h, s_ref[...], w0_ref[...], b0_ref[...], w1_ref[...],
               b1_ref[...], ew_ref[...], m0_ref, m1_ref, base_ref)


def _t_mid_body(hp_ref, ag_ref, g_ref, be_ref, w0_ref, b0_ref, w1_ref,
                b1_ref, s_ref, ew_ref, h_ref, m0_ref, m1_ref, base_ref):
    x = hp_ref[...] + jnp.maximum(ag_ref[...], 0.0)
    h = _ln(x, g_ref[...], be_ref[...])
    h_ref[...] = h
    _mats_body(h, s_ref[...], w0_ref[...], b0_ref[...], w1_ref[...],
               b1_ref[...], ew_ref[...], m0_ref, m1_ref, base_ref)


def _t_out_body(hp_ref, ag_ref, g_ref, be_ref, h_ref):
    x = hp_ref[...] + jnp.maximum(ag_ref[...], 0.0)
    h_ref[...] = _ln(x, g_ref[...], be_ref[...])


_row_spec = pl.BlockSpec((TBLK, D), lambda i: (i, 0))
_w_spec = pl.BlockSpec((D, D), lambda i: (0, 0))
_b_spec = pl.BlockSpec((1, D), lambda i: (0, 0))
_GRID = (NPAD // TBLK,)
_sds = jax.ShapeDtypeStruct((NPAD, D), f32)

_t_in = pl.pallas_call(
    _t_in_body, grid=_GRID,
    in_specs=[_row_spec, _w_spec, _b_spec, _w_spec, _b_spec, _w_spec,
              _b_spec, _row_spec, _w_spec],
    out_specs=[_row_spec] * 4, out_shape=[_sds] * 4)

_t_mid = pl.pallas_call(
    _t_mid_body, grid=_GRID,
    in_specs=[_row_spec, _row_spec, _b_spec, _b_spec, _w_spec, _b_spec,
              _w_spec, _b_spec, _row_spec, _w_spec],
    out_specs=[_row_spec] * 4, out_shape=[_sds] * 4)

_t_out = pl.pallas_call(
    _t_out_body, grid=_GRID,
    in_specs=[_row_spec, _row_spec, _b_spec, _b_spec],
    out_specs=_row_spec, out_shape=_sds)


def kernel(node_feat, edge_index_0, edge_attr_0, edge_index_1, edge_attr_1,
           params):
    del edge_attr_1
    nf = jnp.concatenate(
        [node_feat[0], jnp.zeros((NPAD - N, D), f32)], axis=0)
    pad_src = jnp.zeros((EPAD - E,), i32)
    pad_dst = jnp.full((EPAD - E,), NPAD - 1, i32)
    src_all = jnp.concatenate([
        edge_index_0[0], pad_src, edge_index_1[0], pad_src])
    dst_all = jnp.concatenate([
        edge_index_0[1], pad_dst, edge_index_1[1], pad_dst])
    a_rows = jnp.concatenate(
        [edge_attr_0, jnp.ones((E, 1), f32), jnp.zeros((E, D - 5), f32)],
        axis=1)
    a_rows = jnp.concatenate([a_rows, jnp.zeros((EPAD - E, D), f32)],
                             axis=0)
    zrows = jnp.zeros((RPT, D), f32)

    bin_k, s_k, agg_k = _sc_kernels()
    bin_src, bin_dloc, bin_eid, counts = bin_k(src_all, dst_all)
    bin_src = bin_src.reshape(-1, BATCH)
    bin_dloc = bin_dloc.reshape(-1, BATCH)
    bin_eid = bin_eid.reshape(-1, BATCH)
    s_mat = s_k(a_rows, bin_eid, bin_dloc, counts, zrows)

    layers = params["layers"]

    def ew_mat(layer):
        return jnp.concatenate(
            [layer["edge_W"][0], layer["edge_b"][0].reshape(1, D),
             jnp.zeros((D - 5, D), f32)], axis=0)

    l0 = layers[0]
    h, m0, m1, base = _t_in(
        nf, params["input_W"], params["input_b"].reshape(1, D),
        l0["node_W"][0], l0["node_b"][0].reshape(1, D),
        l0["node_W"][1], l0["node_b"][1].reshape(1, D),
        s_mat, ew_mat(l0))

    for li in range(NL):
        agg = agg_k(m0, m1, base, bin_src, bin_dloc, counts)
        lg = layers[li]["gamma"].reshape(1, D)
        lb = layers[li]["beta"].reshape(1, D)
        if li < NL - 1:
            nxt = layers[li + 1]
            h, m0, m1, base = _t_mid(
                h, agg, lg, lb,
                nxt["node_W"][0], nxt["node_b"][0].reshape(1, D),
                nxt["node_W"][1], nxt["node_b"][1].reshape(1, D),
                s_mat, ew_mat(nxt))
        else:
            h = _t_out(h, agg, lg, lb)

    return h[:N].reshape(1, N, D)


# speedup vs baseline: 1.0102x; 1.0102x over previous
import functools

import jax
import jax.numpy as jnp
from jax import lax
from jax.experimental import pallas as pl
from jax.experimental.pallas import tpu as pltpu
from jax.experimental.pallas import tpu_sc as plsc

N = 50000
D = 128
E = 300000
R = 2
NL = 2

NC, NS, LANES = 2, 16, 16
NW = NC * NS

NCHUNK = 8
CHUNK = 6272
NPAD = NCHUNK * CHUNK
CPSC = NCHUNK // NC
DUMP = CHUNK
RPT = CHUNK // NS

EPW = 9376
EPAD = EPW * NW
BATCH = 128
NBAT = 80
CAP = NBAT * BATCH
NPIPE = 3

TBLK = 512

f32 = jnp.float32
i32 = jnp.int32


@functools.lru_cache(maxsize=1)
def _sc_kernels():
    mesh = plsc.VectorSubcoreMesh(core_axis_name="c", subcore_axis_name="s",
                                  num_cores=NC, num_subcores=NS)
    cparams = pltpu.CompilerParams(needs_layout_passes=False)
    bin_k = functools.partial(
        pl.kernel,
        out_type=[
            jax.ShapeDtypeStruct((R * NCHUNK * NW * CAP,), i32),
            jax.ShapeDtypeStruct((R * NCHUNK * NW * CAP,), i32),
            jax.ShapeDtypeStruct((R * NCHUNK * NW * CAP,), i32),
            jax.ShapeDtypeStruct((R * NW * NCHUNK * LANES,), i32),
        ],
        mesh=mesh,
        compiler_params=cparams,
        scratch_types=[
            pltpu.VMEM((EPW,), i32),
            pltpu.VMEM((EPW,), i32),
            pltpu.VMEM((CAP,), i32),
            pltpu.VMEM((CAP,), i32),
            pltpu.VMEM((CAP,), i32),
            pltpu.VMEM((NCHUNK * LANES,), i32),
        ],
    )(_bin_body)
    s_k = functools.partial(
        pl.kernel,
        out_type=jax.ShapeDtypeStruct((NPAD, D), f32),
        mesh=mesh,
        compiler_params=cparams,
        scratch_types=[
            pltpu.VMEM((NPIPE * BATCH,), i32),
            [pltpu.VMEM((BATCH,), i32)] * NPIPE,
            pltpu.VMEM((NPIPE, BATCH, D), f32),
            pltpu.VMEM((NC * NCHUNK * LANES,), i32),
            pltpu.VMEM_SHARED((CHUNK + 1, D), f32),
            [pltpu.SemaphoreType.DMA] * NPIPE,
        ],
    )(_s_body)
    agg_k = functools.partial(
        pl.kernel,
        out_type=jax.ShapeDtypeStruct((NPAD, D), f32),
        mesh=mesh,
        compiler_params=cparams,
        scratch_types=[
            pltpu.VMEM((NPIPE * BATCH,), i32),
            [pltpu.VMEM((BATCH,), i32)] * NPIPE,
            pltpu.VMEM((NPIPE, BATCH, D), f32),
            pltpu.VMEM((R * NC * NCHUNK * LANES,), i32),
            pltpu.VMEM_SHARED((CHUNK + 1, D), f32),
            [pltpu.SemaphoreType.DMA] * NPIPE,
        ],
    )(_agg_body)
    return bin_k, s_k, agg_k


def _bin_body(src_hbm, dst_hbm, bin_src, bin_dloc, bin_eid, counts,
              st_s, st_d, seg_s, seg_dl, seg_e, cnt_st):
    cid_ax = lax.axis_index("c")
    sid_ax = lax.axis_index("s")
    wid = sid_ax * NC + cid_ax
    lanes = lax.iota(i32, LANES)
    zeros16 = jnp.zeros((LANES,), i32)
    dump16 = jnp.full((LANES,), DUMP, i32)
    epad16 = jnp.full((LANES,), E, i32)
    ebase = wid * EPW

    for r in range(R):
        pltpu.sync_copy(src_hbm.at[pl.ds(r * EPAD + ebase, EPW)], st_s)
        pltpu.sync_copy(dst_hbm.at[pl.ds(r * EPAD + ebase, EPW)], st_d)
        for c in range(NCHUNK):
            lo, hi = c * CHUNK, (c + 1) * CHUNK

            def body(i, off):
                s = st_s[pl.ds(i * LANES, LANES)]
                d = st_d[pl.ds(i * LANES, LANES)]
                eid = (ebase + i * LANES) + lanes
                m = (d >= lo) & (d < hi)
                plsc.store_compressed(seg_s.at[pl.ds(off, LANES)], s,
                                      mask=m)
                plsc.store_compressed(seg_dl.at[pl.ds(off, LANES)], d - lo,
                                      mask=m)
                plsc.store_compressed(seg_e.at[pl.ds(off, LANES)], eid,
                                      mask=m)
                return off + plsc.all_reduce_population_count(m)[0]

            off = lax.fori_loop(0, EPW // LANES, body, jnp.int32(0))
            for k in range(NPIPE * BATCH // LANES):
                seg_s[pl.ds(off + k * LANES, LANES)] = zeros16
                seg_dl[pl.ds(off + k * LANES, LANES)] = dump16
                seg_e[pl.ds(off + k * LANES, LANES)] = epad16
            cnt_st[pl.ds(c * LANES, LANES)] = jnp.full((LANES,), off, i32)
            row = (r * NCHUNK + c) * NW + wid
            pltpu.sync_copy(seg_s, bin_src.at[pl.ds(row * CAP, CAP)])
            pltpu.sync_copy(seg_dl, bin_dloc.at[pl.ds(row * CAP, CAP)])
            pltpu.sync_copy(seg_e, bin_eid.at[pl.ds(row * CAP, CAP)])
        pltpu.sync_copy(
            cnt_st,
            counts.at[pl.ds((r * NW + wid) * NCHUNK * LANES,
                            NCHUNK * LANES)])


def _s_body(a_hbm, bin_eid, bin_dloc, counts, zrows,
            s_out, eidv, dlocv, arows, cnt, acc, sems):
    cid_ax = lax.axis_index("c")
    sid_ax = lax.axis_index("s")
    pltpu.sync_copy(
        counts.at[pl.ds(2 * sid_ax * NCHUNK * LANES, 2 * NCHUNK * LANES)],
        cnt)

    for cc in range(CPSC):
        chunk = cid_ax + NC * cc
        pltpu.sync_copy(zrows, acc.at[pl.ds(sid_ax * RPT, RPT)])
        plsc.subcore_barrier()
        for sl in range(2):
            seg = 2 * sid_ax + sl
            n = cnt[pl.ds((sl * NCHUNK + chunk) * LANES, LANES)][0]
            base_off = (chunk * NW + seg) * CAP
            ng = (n + NPIPE * BATCH - 1) // (NPIPE * BATCH)

            def bbody(g, _):
                e0 = base_off + g * NPIPE * BATCH
                pltpu.sync_copy(bin_eid.at[pl.ds(e0, NPIPE * BATCH)],
                                eidv)
                cps = []
                for k in range(NPIPE):
                    pltpu.sync_copy(
                        bin_dloc.at[pl.ds(e0 + k * BATCH, BATCH)],
                        dlocv[k])
                    cps.append(pltpu.async_copy(
                        a_hbm.at[eidv.at[pl.ds(k * BATCH, BATCH)]],
                        arows.at[k], sems[k]))
                for k in range(NPIPE):
                    cps[k].wait()
                    pltpu.sync_copy(arows.at[k], acc.at[dlocv[k]],
                                    add=True)
                return 0

            lax.fori_loop(0, ng, bbody, 0)
        plsc.subcore_barrier()
        pltpu.sync_copy(
            acc.at[pl.ds(sid_ax * RPT, RPT)],
            s_out.at[pl.ds(chunk * CHUNK + sid_ax * RPT, RPT)])
        plsc.subcore_barrier()


def _agg_body(m0_hbm, m1_hbm, base_hbm, bin_src, bin_dloc, counts,
              agg, srcv, dlocv, rows, cnt, acc, sems):
    cid_ax = lax.axis_index("c")
    sid_ax = lax.axis_index("s")
    for r in range(R):
        pltpu.sync_copy(
            counts.at[pl.ds((r * NW + 2 * sid_ax) * NCHUNK * LANES,
                            2 * NCHUNK * LANES)],
            cnt.at[pl.ds(r * 2 * NCHUNK * LANES, 2 * NCHUNK * LANES)])

    for cc in range(CPSC):
        chunk = cid_ax + NC * cc
        rowbase = chunk * CHUNK + sid_ax * RPT
        pltpu.sync_copy(base_hbm.at[pl.ds(rowbase, RPT)],
                        acc.at[pl.ds(sid_ax * RPT, RPT)])
        plsc.subcore_barrier()
        for r in range(R):
            m_hbm = m0_hbm if r == 0 else m1_hbm
            for sl in range(2):
                seg = 2 * sid_ax + sl
                n = cnt[pl.ds(((r * 2 + sl) * NCHUNK + chunk) * LANES,
                              LANES)][0]
                base_off = ((r * NCHUNK + chunk) * NW + seg) * CAP
                ng = (n + NPIPE * BATCH - 1) // (NPIPE * BATCH)

                def gbody(g, _, m_hbm=m_hbm, base_off=base_off):
                    e0 = base_off + g * NPIPE * BATCH
                    pltpu.sync_copy(bin_src.at[pl.ds(e0, NPIPE * BATCH)],
                                    srcv)
                    cps = []
                    for k in range(NPIPE):
                        pltpu.sync_copy(
                            bin_dloc.at[pl.ds(e0 + k * BATCH, BATCH)],
                            dlocv[k])
                        cps.append(pltpu.async_copy(
                            m_hbm.at[srcv.at[pl.ds(k * BATCH, BATCH)]],
                            rows.at[k], sems[k]))
                    for k in range(NPIPE):
                        cps[k].wait()
                        pltpu.sync_copy(rows.at[k], acc.at[dlocv[k]],
                                        add=True)
                    return 0

                lax.fori_loop(0, ng, gbody, 0)
        plsc.subcore_barrier()
        pltpu.sync_copy(acc.at[pl.ds(sid_ax * RPT, RPT)],
                        agg.at[pl.ds(rowbase, RPT)])
        plsc.subcore_barrier()


def _ln(x, g, b):
    m = jnp.mean(x, axis=-1, keepdims=True)
    xc = x - m
    v = jnp.mean(xc * xc, axis=-1, keepdims=True)
    return g * xc * lax.rsqrt(v + 1e-5) + b


def _mats_body(h, s, w0, b0, w1, b1, ew, m0_ref, m1_ref, base_ref):
    m0_ref[...] = jnp.dot(h, w0, preferred_element_type=f32) + b0
    m1_ref[...] = jnp.dot(h, w1, preferred_element_type=f32) + b1
    base_ref[...] = jnp.dot(s, ew, preferred_element_type=f32)


def _t_in_body(nf_ref, iw_ref, ib_ref, w0_ref, b0_ref, w1_ref, b1_ref,
               s_ref, ew_ref, h_ref, m0_ref, m1_ref, base_ref):
    h = jnp.dot(nf_ref[...], iw_ref[...], preferred_element_type=f32) \
        + ib_ref[...]
    h_ref[...] = h
    _mats_body(h, s_ref[...], w0_ref[...], b0_ref[...], w1_ref[...],
               b1_ref[...], ew_ref[...], m0_ref, m1_ref, base_ref)


def _t_mid_body(hp_ref, ag_ref, g_ref, be_ref, w0_ref, b0_ref, w1_ref,
                b1_ref, s_ref, ew_ref, h_ref, m0_ref, m1_ref, base_ref):
    x = hp_ref[...] + jnp.maximum(ag_ref[...], 0.0)
    h = _ln(x, g_ref[...], be_ref[...])
    h_ref[...] = h
    _mats_body(h, s_ref[...], w0_ref[...], b0_ref[...], w1_ref[...],
               b1_ref[...], ew_ref[...], m0_ref, m1_ref, base_ref)


def _t_out_body(hp_ref, ag_ref, g_ref, be_ref, h_ref):
    x = hp_ref[...] + jnp.maximum(ag_ref[...], 0.0)
    h_ref[...] = _ln(x, g_ref[...], be_ref[...])


_row_spec = pl.BlockSpec((TBLK, D), lambda i: (i, 0))
_w_spec = pl.BlockSpec((D, D), lambda i: (0, 0))
_b_spec = pl.BlockSpec((1, D), lambda i: (0, 0))
_GRID = (NPAD // TBLK,)
_sds = jax.ShapeDtypeStruct((NPAD, D), f32)

_t_in = pl.pallas_call(
    _t_in_body, grid=_GRID,
    in_specs=[_row_spec, _w_spec, _b_spec, _w_spec, _b_spec, _w_spec,
              _b_spec, _row_spec, _w_spec],
    out_specs=[_row_spec] * 4, out_shape=[_sds] * 4)

_t_mid = pl.pallas_call(
    _t_mid_body, grid=_GRID,
    in_specs=[_row_spec, _row_spec, _b_spec, _b_spec, _w_spec, _b_spec,
              _w_spec, _b_spec, _row_spec, _w_spec],
    out_specs=[_row_spec] * 4, out_shape=[_sds] * 4)

_t_out = pl.pallas_call(
    _t_out_body, grid=_GRID,
    in_specs=[_row_spec, _row_spec, _b_spec, _b_spec],
    out_specs=_row_spec, out_shape=_sds)


def kernel(node_feat, edge_index_0, edge_attr_0, edge_index_1, edge_attr_1,
           params):
    del edge_attr_1
    nf = jnp.concatenate(
        [node_feat[0], jnp.zeros((NPAD - N, D), f32)], axis=0)
    pad_src = jnp.zeros((EPAD - E,), i32)
    pad_dst = jnp.full((EPAD - E,), NPAD - 1, i32)
    src_all = jnp.concatenate([
        edge_index_0[0], pad_src, edge_index_1[0], pad_src])
    dst_all = jnp.concatenate([
        edge_index_0[1], pad_dst, edge_index_1[1], pad_dst])
    a_rows = jnp.concatenate(
        [edge_attr_0, jnp.ones((E, 1), f32), jnp.zeros((E, D - 5), f32)],
        axis=1)
    a_rows = jnp.concatenate([a_rows, jnp.zeros((EPAD - E, D), f32)],
                             axis=0)
    zrows = jnp.zeros((RPT, D), f32)

    bin_k, s_k, agg_k = _sc_kernels()
    bin_src, bin_dloc, bin_eid, counts = bin_k(src_all, dst_all)
    s_mat = s_k(a_rows, bin_eid, bin_dloc, counts, zrows)

    layers = params["layers"]

    def ew_mat(layer):
        return jnp.concatenate(
            [layer["edge_W"][0], layer["edge_b"][0].reshape(1, D),
             jnp.zeros((D - 5, D), f32)], axis=0)

    l0 = layers[0]
    h, m0, m1, base = _t_in(
        nf, params["input_W"], params["input_b"].reshape(1, D),
        l0["node_W"][0], l0["node_b"][0].reshape(1, D),
        l0["node_W"][1], l0["node_b"][1].reshape(1, D),
        s_mat, ew_mat(l0))

    for li in range(NL):
        agg = agg_k(m0, m1, base, bin_src, bin_dloc, counts)
        lg = layers[li]["gamma"].reshape(1, D)
        lb = layers[li]["beta"].reshape(1, D)
        if li < NL - 1:
            nxt = layers[li + 1]
            h, m0, m1, base = _t_mid(
                h, agg, lg, lb,
                nxt["node_W"][0], nxt["node_b"][0].reshape(1, D),
                nxt["node_W"][1], nxt["node_b"][1].reshape(1, D),
                s_mat, ew_mat(nxt))
        else:
            h = _t_out(h, agg, lg, lb)

    return h[:N].reshape(1, N, D)


# speedup vs baseline: 2.5648x; 2.5389x over previous
import functools

import jax
import jax.numpy as jnp
from jax import lax
from jax.experimental import pallas as pl
from jax.experimental.pallas import tpu as pltpu
from jax.experimental.pallas import tpu_sc as plsc

N = 50000
D = 128
E = 300000
R = 2
NL = 2

NC, NS, LANES = 2, 16, 16
NW = NC * NS

NCHUNK = 8
CHUNK = 6272
NPAD = NCHUNK * CHUNK
CPSC = NCHUNK // NC
DUMP = CHUNK
RPT = CHUNK // NS

EPW = 9376
EPAD = EPW * NW
BATCH = 128
NBAT = 80
CAP = NBAT * BATCH
NPIPE = 1

TBLK = 512

f32 = jnp.float32
i32 = jnp.int32


@functools.lru_cache(maxsize=1)
def _sc_kernels():
    mesh = plsc.VectorSubcoreMesh(core_axis_name="c", subcore_axis_name="s",
                                  num_cores=NC, num_subcores=NS)
    cparams = pltpu.CompilerParams(needs_layout_passes=False)
    bin_k = functools.partial(
        pl.kernel,
        out_type=[
            jax.ShapeDtypeStruct((R * NCHUNK * NW * CAP,), i32),
            jax.ShapeDtypeStruct((R * NCHUNK * NW * CAP,), i32),
            jax.ShapeDtypeStruct((R * NCHUNK * NW * CAP,), i32),
            jax.ShapeDtypeStruct((R * NW * NCHUNK * LANES,), i32),
        ],
        mesh=mesh,
        compiler_params=cparams,
        scratch_types=[
            pltpu.VMEM((EPW,), i32),
            pltpu.VMEM((EPW,), i32),
            pltpu.VMEM((CAP,), i32),
            pltpu.VMEM((CAP,), i32),
            pltpu.VMEM((CAP,), i32),
            pltpu.VMEM((NCHUNK * LANES,), i32),
        ],
    )(_bin_body)
    s_k = functools.partial(
        pl.kernel,
        out_type=jax.ShapeDtypeStruct((NPAD, D), f32),
        mesh=mesh,
        compiler_params=cparams,
        scratch_types=[
            pltpu.VMEM((NPIPE * BATCH,), i32),
            [pltpu.VMEM((BATCH,), i32)] * NPIPE,
            pltpu.VMEM((NPIPE, BATCH, D), f32),
            pltpu.VMEM((NC * NCHUNK * LANES,), i32),
            pltpu.VMEM_SHARED((CHUNK + 1, D), f32),
            [pltpu.SemaphoreType.DMA] * NPIPE,
        ],
    )(_s_body)
    agg_k = functools.partial(
        pl.kernel,
        out_type=jax.ShapeDtypeStruct((NPAD, D), f32),
        mesh=mesh,
        compiler_params=cparams,
        scratch_types=[
            pltpu.VMEM((NPIPE * BATCH,), i32),
            [pltpu.VMEM((BATCH,), i32)] * NPIPE,
            pltpu.VMEM((NPIPE, BATCH, D), f32),
            pltpu.VMEM((R * NC * NCHUNK * LANES,), i32),
            pltpu.VMEM_SHARED((CHUNK + 1, D), f32),
            [pltpu.SemaphoreType.DMA] * NPIPE,
        ],
    )(_agg_body)
    return bin_k, s_k, agg_k


def _bin_body(src_hbm, dst_hbm, bin_src, bin_dloc, bin_eid, counts,
              st_s, st_d, seg_s, seg_dl, seg_e, cnt_st):
    cid_ax = lax.axis_index("c")
    sid_ax = lax.axis_index("s")
    wid = sid_ax * NC + cid_ax
    lanes = lax.iota(i32, LANES)
    zeros16 = jnp.zeros((LANES,), i32)
    dump16 = jnp.full((LANES,), DUMP, i32)
    epad16 = jnp.full((LANES,), E, i32)
    ebase = wid * EPW

    for r in range(R):
        pltpu.sync_copy(src_hbm.at[pl.ds(r * EPAD + ebase, EPW)], st_s)
        pltpu.sync_copy(dst_hbm.at[pl.ds(r * EPAD + ebase, EPW)], st_d)
        for c in range(NCHUNK):
            lo, hi = c * CHUNK, (c + 1) * CHUNK

            def body(i, off):
                s = st_s[pl.ds(i * LANES, LANES)]
                d = st_d[pl.ds(i * LANES, LANES)]
                eid = (ebase + i * LANES) + lanes
                m = (d >= lo) & (d < hi)
                plsc.store_compressed(seg_s.at[pl.ds(off, LANES)], s,
                                      mask=m)
                plsc.store_compressed(seg_dl.at[pl.ds(off, LANES)], d - lo,
                                      mask=m)
                plsc.store_compressed(seg_e.at[pl.ds(off, LANES)], eid,
                                      mask=m)
                return off + plsc.all_reduce_population_count(m)[0]

            off = lax.fori_loop(0, EPW // LANES, body, jnp.int32(0))
            for k in range(NPIPE * BATCH // LANES):
                seg_s[pl.ds(off + k * LANES, LANES)] = zeros16
                seg_dl[pl.ds(off + k * LANES, LANES)] = dump16
                seg_e[pl.ds(off + k * LANES, LANES)] = epad16
            cnt_st[pl.ds(c * LANES, LANES)] = jnp.full((LANES,), off, i32)
            row = (r * NCHUNK + c) * NW + wid
            pltpu.sync_copy(seg_s, bin_src.at[pl.ds(row * CAP, CAP)])
            pltpu.sync_copy(seg_dl, bin_dloc.at[pl.ds(row * CAP, CAP)])
            pltpu.sync_copy(seg_e, bin_eid.at[pl.ds(row * CAP, CAP)])
        pltpu.sync_copy(
            cnt_st,
            counts.at[pl.ds((r * NW + wid) * NCHUNK * LANES,
                            NCHUNK * LANES)])


def _s_body(a_hbm, bin_eid, bin_dloc, counts, zrows,
            s_out, eidv, dlocv, arows, cnt, acc, sems):
    cid_ax = lax.axis_index("c")
    sid_ax = lax.axis_index("s")
    pltpu.sync_copy(
        counts.at[pl.ds(2 * sid_ax * NCHUNK * LANES, 2 * NCHUNK * LANES)],
        cnt)

    for cc in range(CPSC):
        chunk = cid_ax + NC * cc
        pltpu.sync_copy(zrows, acc.at[pl.ds(sid_ax * RPT, RPT)])
        plsc.subcore_barrier()
        for sl in range(2):
            seg = 2 * sid_ax + sl
            n = cnt[pl.ds((sl * NCHUNK + chunk) * LANES, LANES)][0]
            base_off = (chunk * NW + seg) * CAP
            ng = (n + NPIPE * BATCH - 1) // (NPIPE * BATCH)

            def bbody(g, _):
                e0 = base_off + g * NPIPE * BATCH
                pltpu.sync_copy(bin_eid.at[pl.ds(e0, NPIPE * BATCH)],
                                eidv)
                cps = []
                for k in range(NPIPE):
                    pltpu.sync_copy(
                        bin_dloc.at[pl.ds(e0 + k * BATCH, BATCH)],
                        dlocv[k])
                    cps.append(pltpu.async_copy(
                        a_hbm.at[eidv.at[pl.ds(k * BATCH, BATCH)]],
                        arows.at[k], sems[k]))
                for k in range(NPIPE):
                    cps[k].wait()
                    pltpu.sync_copy(arows.at[k], acc.at[dlocv[k]],
                                    add=True)
                return 0

            lax.fori_loop(0, ng, bbody, 0)
        plsc.subcore_barrier()
        pltpu.sync_copy(
            acc.at[pl.ds(sid_ax * RPT, RPT)],
            s_out.at[pl.ds(chunk * CHUNK + sid_ax * RPT, RPT)])
        plsc.subcore_barrier()


def _agg_body(m0_hbm, m1_hbm, base_hbm, bin_src, bin_dloc, counts,
              agg, srcv, dlocv, rows, cnt, acc, sems):
    cid_ax = lax.axis_index("c")
    sid_ax = lax.axis_index("s")
    for r in range(R):
        pltpu.sync_copy(
            counts.at[pl.ds((r * NW + 2 * sid_ax) * NCHUNK * LANES,
                            2 * NCHUNK * LANES)],
            cnt.at[pl.ds(r * 2 * NCHUNK * LANES, 2 * NCHUNK * LANES)])

    for cc in range(CPSC):
        chunk = cid_ax + NC * cc
        rowbase = chunk * CHUNK + sid_ax * RPT
        pltpu.sync_copy(base_hbm.at[pl.ds(rowbase, RPT)],
                        acc.at[pl.ds(sid_ax * RPT, RPT)])
        plsc.subcore_barrier()
        for r in range(R):
            m_hbm = m0_hbm if r == 0 else m1_hbm
            for sl in range(2):
                seg = 2 * sid_ax + sl
                n = cnt[pl.ds(((r * 2 + sl) * NCHUNK + chunk) * LANES,
                              LANES)][0]
                base_off = ((r * NCHUNK + chunk) * NW + seg) * CAP
                ng = (n + NPIPE * BATCH - 1) // (NPIPE * BATCH)

                def gbody(g, _, m_hbm=m_hbm, base_off=base_off):
                    e0 = base_off + g * NPIPE * BATCH
                    pltpu.sync_copy(bin_src.at[pl.ds(e0, NPIPE * BATCH)],
                                    srcv)
                    cps = []
                    for k in range(NPIPE):
                        pltpu.sync_copy(
                            bin_dloc.at[pl.ds(e0 + k * BATCH, BATCH)],
                            dlocv[k])
                        cps.append(pltpu.async_copy(
                            m_hbm.at[srcv.at[pl.ds(k * BATCH, BATCH)]],
                            rows.at[k], sems[k]))
                    for k in range(NPIPE):
                        cps[k].wait()
                        pltpu.sync_copy(rows.at[k], acc.at[dlocv[k]],
                                        add=True)
                    return 0

                lax.fori_loop(0, ng, gbody, 0)
        plsc.subcore_barrier()
        pltpu.sync_copy(acc.at[pl.ds(sid_ax * RPT, RPT)],
                        agg.at[pl.ds(rowbase, RPT)])
        plsc.subcore_barrier()


def _ln(x, g, b):
    m = jnp.mean(x, axis=-1, keepdims=True)
    xc = x - m
    v = jnp.mean(xc * xc, axis=-1, keepdims=True)
    return g * xc * lax.rsqrt(v + 1e-5) + b


def _mats_body(h, s, w0, b0, w1, b1, ew, m0_ref, m1_ref, base_ref):
    m0_ref[...] = jnp.dot(h, w0, preferred_element_type=f32) + b0
    m1_ref[...] = jnp.dot(h, w1, preferred_element_type=f32) + b1
    base_ref[...] = jnp.dot(s, ew, preferred_element_type=f32)


def _t_in_body(nf_ref, iw_ref, ib_ref, w0_ref, b0_ref, w1_ref, b1_ref,
               s_ref, ew_ref, h_ref, m0_ref, m1_ref, base_ref):
    h = jnp.dot(nf_ref[...], iw_ref[...], preferred_element_type=f32) \
        + ib_ref[...]
    h_ref[...] = h
    _mats_body(h, s_ref[...], w0_ref[...], b0_ref[...], w1_ref[...],
               b1_ref[...], ew_ref[...], m0_ref, m1_ref, base_ref)


def _t_mid_body(hp_ref, ag_ref, g_ref, be_ref, w0_ref, b0_ref, w1_ref,
                b1_ref, s_ref, ew_ref, h_ref, m0_ref, m1_ref, base_ref):
    x = hp_ref[...] + jnp.maximum(ag_ref[...], 0.0)
    h = _ln(x, g_ref[...], be_ref[...])
    h_ref[...] = h
    _mats_body(h, s_ref[...], w0_ref[...], b0_ref[...], w1_ref[...],
               b1_ref[...], ew_ref[...], m0_ref, m1_ref, base_ref)


def _t_out_body(hp_ref, ag_ref, g_ref, be_ref, h_ref):
    x = hp_ref[...] + jnp.maximum(ag_ref[...], 0.0)
    h_ref[...] = _ln(x, g_ref[...], be_ref[...])


_row_spec = pl.BlockSpec((TBLK, D), lambda i: (i, 0))
_w_spec = pl.BlockSpec((D, D), lambda i: (0, 0))
_b_spec = pl.BlockSpec((1, D), lambda i: (0, 0))
_GRID = (NPAD // TBLK,)
_sds = jax.ShapeDtypeStruct((NPAD, D), f32)

_t_in = pl.pallas_call(
    _t_in_body, grid=_GRID,
    in_specs=[_row_spec, _w_spec, _b_spec, _w_spec, _b_spec, _w_spec,
              _b_spec, _row_spec, _w_spec],
    out_specs=[_row_spec] * 4, out_shape=[_sds] * 4)

_t_mid = pl.pallas_call(
    _t_mid_body, grid=_GRID,
    in_specs=[_row_spec, _row_spec, _b_spec, _b_spec, _w_spec, _b_spec,
              _w_spec, _b_spec, _row_spec, _w_spec],
    out_specs=[_row_spec] * 4, out_shape=[_sds] * 4)

_t_out = pl.pallas_call(
    _t_out_body, grid=_GRID,
    in_specs=[_row_spec, _row_spec, _b_spec, _b_spec],
    out_specs=_row_spec, out_shape=_sds)


def kernel(node_feat, edge_index_0, edge_attr_0, edge_index_1, edge_attr_1,
           params):
    del edge_attr_1
    nf = jnp.concatenate(
        [node_feat[0], jnp.zeros((NPAD - N, D), f32)], axis=0)
    pad_src = jnp.zeros((EPAD - E,), i32)
    pad_dst = jnp.full((EPAD - E,), NPAD - 1, i32)
    src_all = jnp.concatenate([
        edge_index_0[0], pad_src, edge_index_1[0], pad_src])
    dst_all = jnp.concatenate([
        edge_index_0[1], pad_dst, edge_index_1[1], pad_dst])
    a_rows = jnp.concatenate(
        [edge_attr_0, jnp.ones((E, 1), f32), jnp.zeros((E, D - 5), f32)],
        axis=1)
    a_rows = jnp.concatenate([a_rows, jnp.zeros((EPAD - E, D), f32)],
                             axis=0)
    zrows = jnp.zeros((RPT, D), f32)

    bin_k, s_k, agg_k = _sc_kernels()
    bin_src, bin_dloc, bin_eid, counts = bin_k(src_all, dst_all)
    s_mat = s_k(a_rows, bin_eid, bin_dloc, counts, zrows)

    layers = params["layers"]

    def ew_mat(layer):
        return jnp.concatenate(
            [layer["edge_W"][0], layer["edge_b"][0].reshape(1, D),
             jnp.zeros((D - 5, D), f32)], axis=0)

    l0 = layers[0]
    h, m0, m1, base = _t_in(
        nf, params["input_W"], params["input_b"].reshape(1, D),
        l0["node_W"][0], l0["node_b"][0].reshape(1, D),
        l0["node_W"][1], l0["node_b"][1].reshape(1, D),
        s_mat, ew_mat(l0))

    for li in range(NL):
        agg = agg_k(m0, m1, base, bin_src, bin_dloc, counts)
        lg = layers[li]["gamma"].reshape(1, D)
        lb = layers[li]["beta"].reshape(1, D)
        if li < NL - 1:
            nxt = layers[li + 1]
            h, m0, m1, base = _t_mid(
                h, agg, lg, lb,
                nxt["node_W"][0], nxt["node_b"][0].reshape(1, D),
                nxt["node_W"][1], nxt["node_b"][1].reshape(1, D),
                s_mat, ew_mat(nxt))
        else:
            h = _t_out(h, agg, lg, lb)

    return h[:N].reshape(1, N, D)


# speedup vs baseline: 3.9738x; 1.5494x over previous
import functools

import jax
import jax.numpy as jnp
from jax import lax
from jax.experimental import pallas as pl
from jax.experimental.pallas import tpu as pltpu
from jax.experimental.pallas import tpu_sc as plsc

N = 50000
D = 128
E = 300000
R = 2
NL = 2

NC, NS, LANES = 2, 16, 16
NW = NC * NS

NCHUNK = 4
CHUNK = 12544
NPAD = NCHUNK * CHUNK
CPSC = NCHUNK // NC
DUMP = CHUNK
RPT = CHUNK // NS

EPW = 9376
EPAD = EPW * NW
BATCH = 128
NBAT = 80
CAP = NBAT * BATCH
NPIPE = 1

TBLK = 512

f32 = jnp.float32
i32 = jnp.int32


@functools.lru_cache(maxsize=1)
def _sc_kernels():
    mesh = plsc.VectorSubcoreMesh(core_axis_name="c", subcore_axis_name="s",
                                  num_cores=NC, num_subcores=NS)
    cparams = pltpu.CompilerParams(needs_layout_passes=False)
    bin_k = functools.partial(
        pl.kernel,
        out_type=[
            jax.ShapeDtypeStruct((R * NCHUNK * NW * CAP,), i32),
            jax.ShapeDtypeStruct((R * NCHUNK * NW * CAP,), i32),
            jax.ShapeDtypeStruct((R * NCHUNK * NW * CAP,), i32),
            jax.ShapeDtypeStruct((R * NW * NCHUNK * LANES,), i32),
        ],
        mesh=mesh,
        compiler_params=cparams,
        scratch_types=[
            pltpu.VMEM((EPW,), i32),
            pltpu.VMEM((EPW,), i32),
            pltpu.VMEM((CAP,), i32),
            pltpu.VMEM((CAP,), i32),
            pltpu.VMEM((CAP,), i32),
            pltpu.VMEM((NCHUNK * LANES,), i32),
        ],
    )(_bin_body)
    s_k = functools.partial(
        pl.kernel,
        out_type=jax.ShapeDtypeStruct((NPAD, D), f32),
        mesh=mesh,
        compiler_params=cparams,
        scratch_types=[
            pltpu.VMEM((NPIPE * BATCH,), i32),
            [pltpu.VMEM((BATCH,), i32)] * NPIPE,
            pltpu.VMEM((NPIPE, BATCH, D), f32),
            pltpu.VMEM((NC * NCHUNK * LANES,), i32),
            pltpu.VMEM_SHARED((CHUNK + 1, D), f32),
            [pltpu.SemaphoreType.DMA] * NPIPE,
        ],
    )(_s_body)
    agg_k = functools.partial(
        pl.kernel,
        out_type=jax.ShapeDtypeStruct((NPAD, D), f32),
        mesh=mesh,
        compiler_params=cparams,
        scratch_types=[
            pltpu.VMEM((NPIPE * BATCH,), i32),
            [pltpu.VMEM((BATCH,), i32)] * NPIPE,
            pltpu.VMEM((NPIPE, BATCH, D), f32),
            pltpu.VMEM((R * NC * NCHUNK * LANES,), i32),
            pltpu.VMEM_SHARED((CHUNK + 1, D), f32),
            [pltpu.SemaphoreType.DMA] * NPIPE,
        ],
    )(_agg_body)
    return bin_k, s_k, agg_k


def _bin_body(src_hbm, dst_hbm, bin_src, bin_dloc, bin_eid, counts,
              st_s, st_d, seg_s, seg_dl, seg_e, cnt_st):
    cid_ax = lax.axis_index("c")
    sid_ax = lax.axis_index("s")
    wid = sid_ax * NC + cid_ax
    lanes = lax.iota(i32, LANES)
    zeros16 = jnp.zeros((LANES,), i32)
    dump16 = jnp.full((LANES,), DUMP, i32)
    epad16 = jnp.full((LANES,), E, i32)
    ebase = wid * EPW

    for r in range(R):
        pltpu.sync_copy(src_hbm.at[pl.ds(r * EPAD + ebase, EPW)], st_s)
        pltpu.sync_copy(dst_hbm.at[pl.ds(r * EPAD + ebase, EPW)], st_d)
        for c in range(NCHUNK):
            lo, hi = c * CHUNK, (c + 1) * CHUNK

            def body(i, off):
                s = st_s[pl.ds(i * LANES, LANES)]
                d = st_d[pl.ds(i * LANES, LANES)]
                eid = (ebase + i * LANES) + lanes
                m = (d >= lo) & (d < hi)
                plsc.store_compressed(seg_s.at[pl.ds(off, LANES)], s,
                                      mask=m)
                plsc.store_compressed(seg_dl.at[pl.ds(off, LANES)], d - lo,
                                      mask=m)
                plsc.store_compressed(seg_e.at[pl.ds(off, LANES)], eid,
                                      mask=m)
                return off + plsc.all_reduce_population_count(m)[0]

            off = lax.fori_loop(0, EPW // LANES, body, jnp.int32(0))
            for k in range(NPIPE * BATCH // LANES):
                seg_s[pl.ds(off + k * LANES, LANES)] = zeros16
                seg_dl[pl.ds(off + k * LANES, LANES)] = dump16
                seg_e[pl.ds(off + k * LANES, LANES)] = epad16
            cnt_st[pl.ds(c * LANES, LANES)] = jnp.full((LANES,), off, i32)
            row = (r * NCHUNK + c) * NW + wid
            pltpu.sync_copy(seg_s, bin_src.at[pl.ds(row * CAP, CAP)])
            pltpu.sync_copy(seg_dl, bin_dloc.at[pl.ds(row * CAP, CAP)])
            pltpu.sync_copy(seg_e, bin_eid.at[pl.ds(row * CAP, CAP)])
        pltpu.sync_copy(
            cnt_st,
            counts.at[pl.ds((r * NW + wid) * NCHUNK * LANES,
                            NCHUNK * LANES)])


def _s_body(a_hbm, bin_eid, bin_dloc, counts, zrows,
            s_out, eidv, dlocv, arows, cnt, acc, sems):
    cid_ax = lax.axis_index("c")
    sid_ax = lax.axis_index("s")
    pltpu.sync_copy(
        counts.at[pl.ds(2 * sid_ax * NCHUNK * LANES, 2 * NCHUNK * LANES)],
        cnt)

    for cc in range(CPSC):
        chunk = cid_ax + NC * cc
        pltpu.sync_copy(zrows, acc.at[pl.ds(sid_ax * RPT, RPT)])
        plsc.subcore_barrier()
        for sl in range(2):
            seg = 2 * sid_ax + sl
            n = cnt[pl.ds((sl * NCHUNK + chunk) * LANES, LANES)][0]
            base_off = (chunk * NW + seg) * CAP
            ng = (n + NPIPE * BATCH - 1) // (NPIPE * BATCH)

            def bbody(g, _):
                e0 = base_off + g * NPIPE * BATCH
                pltpu.sync_copy(bin_eid.at[pl.ds(e0, NPIPE * BATCH)],
                                eidv)
                cps = []
                for k in range(NPIPE):
                    pltpu.sync_copy(
                        bin_dloc.at[pl.ds(e0 + k * BATCH, BATCH)],
                        dlocv[k])
                    cps.append(pltpu.async_copy(
                        a_hbm.at[eidv.at[pl.ds(k * BATCH, BATCH)]],
                        arows.at[k], sems[k]))
                for k in range(NPIPE):
                    cps[k].wait()
                    pltpu.sync_copy(arows.at[k], acc.at[dlocv[k]],
                                    add=True)
                return 0

            lax.fori_loop(0, ng, bbody, 0)
        plsc.subcore_barrier()
        pltpu.sync_copy(
            acc.at[pl.ds(sid_ax * RPT, RPT)],
            s_out.at[pl.ds(chunk * CHUNK + sid_ax * RPT, RPT)])
        plsc.subcore_barrier()


def _agg_body(m0_hbm, m1_hbm, base_hbm, bin_src, bin_dloc, counts,
              agg, srcv, dlocv, rows, cnt, acc, sems):
    cid_ax = lax.axis_index("c")
    sid_ax = lax.axis_index("s")
    for r in range(R):
        pltpu.sync_copy(
            counts.at[pl.ds((r * NW + 2 * sid_ax) * NCHUNK * LANES,
                            2 * NCHUNK * LANES)],
            cnt.at[pl.ds(r * 2 * NCHUNK * LANES, 2 * NCHUNK * LANES)])

    for cc in range(CPSC):
        chunk = cid_ax + NC * cc
        rowbase = chunk * CHUNK + sid_ax * RPT
        pltpu.sync_copy(base_hbm.at[pl.ds(rowbase, RPT)],
                        acc.at[pl.ds(sid_ax * RPT, RPT)])
        plsc.subcore_barrier()
        for r in range(R):
            m_hbm = m0_hbm if r == 0 else m1_hbm
            for sl in range(2):
                seg = 2 * sid_ax + sl
                n = cnt[pl.ds(((r * 2 + sl) * NCHUNK + chunk) * LANES,
                              LANES)][0]
                base_off = ((r * NCHUNK + chunk) * NW + seg) * CAP
                ng = (n + NPIPE * BATCH - 1) // (NPIPE * BATCH)

                def gbody(g, _, m_hbm=m_hbm, base_off=base_off):
                    e0 = base_off + g * NPIPE * BATCH
                    pltpu.sync_copy(bin_src.at[pl.ds(e0, NPIPE * BATCH)],
                                    srcv)
                    cps = []
                    for k in range(NPIPE):
                        pltpu.sync_copy(
                            bin_dloc.at[pl.ds(e0 + k * BATCH, BATCH)],
                            dlocv[k])
                        cps.append(pltpu.async_copy(
                            m_hbm.at[srcv.at[pl.ds(k * BATCH, BATCH)]],
                            rows.at[k], sems[k]))
                    for k in range(NPIPE):
                        cps[k].wait()
                        pltpu.sync_copy(rows.at[k], acc.at[dlocv[k]],
                                        add=True)
                    return 0

                lax.fori_loop(0, ng, gbody, 0)
        plsc.subcore_barrier()
        pltpu.sync_copy(acc.at[pl.ds(sid_ax * RPT, RPT)],
                        agg.at[pl.ds(rowbase, RPT)])
        plsc.subcore_barrier()


def _ln(x, g, b):
    m = jnp.mean(x, axis=-1, keepdims=True)
    xc = x - m
    v = jnp.mean(xc * xc, axis=-1, keepdims=True)
    return g * xc * lax.rsqrt(v + 1e-5) + b


def _mats_body(h, s, w0, b0, w1, b1, ew, m0_ref, m1_ref, base_ref):
    m0_ref[...] = jnp.dot(h, w0, preferred_element_type=f32) + b0
    m1_ref[...] = jnp.dot(h, w1, preferred_element_type=f32) + b1
    base_ref[...] = jnp.dot(s, ew, preferred_element_type=f32)


def _t_in_body(nf_ref, iw_ref, ib_ref, w0_ref, b0_ref, w1_ref, b1_ref,
               h_ref, m0_ref, m1_ref):
    h = jnp.dot(nf_ref[...], iw_ref[...], preferred_element_type=f32) \
        + ib_ref[...]
    h_ref[...] = h
    m0_ref[...] = jnp.dot(h, w0_ref[...], preferred_element_type=f32) \
        + b0_ref[...]
    m1_ref[...] = jnp.dot(h, w1_ref[...], preferred_element_type=f32) \
        + b1_ref[...]


def _base_body(s_ref, ew_ref, base_ref):
    base_ref[...] = jnp.dot(s_ref[...], ew_ref[...],
                            preferred_element_type=f32)


def _t_mid_body(hp_ref, ag_ref, g_ref, be_ref, w0_ref, b0_ref, w1_ref,
                b1_ref, s_ref, ew_ref, h_ref, m0_ref, m1_ref, base_ref):
    x = hp_ref[...] + jnp.maximum(ag_ref[...], 0.0)
    h = _ln(x, g_ref[...], be_ref[...])
    h_ref[...] = h
    _mats_body(h, s_ref[...], w0_ref[...], b0_ref[...], w1_ref[...],
               b1_ref[...], ew_ref[...], m0_ref, m1_ref, base_ref)


def _t_out_body(hp_ref, ag_ref, g_ref, be_ref, h_ref):
    x = hp_ref[...] + jnp.maximum(ag_ref[...], 0.0)
    h_ref[...] = _ln(x, g_ref[...], be_ref[...])


_row_spec = pl.BlockSpec((TBLK, D), lambda i: (i, 0))
_w_spec = pl.BlockSpec((D, D), lambda i: (0, 0))
_b_spec = pl.BlockSpec((1, D), lambda i: (0, 0))
_GRID = (NPAD // TBLK,)
_sds = jax.ShapeDtypeStruct((NPAD, D), f32)

_t_in = pl.pallas_call(
    _t_in_body, grid=_GRID,
    in_specs=[_row_spec, _w_spec, _b_spec, _w_spec, _b_spec, _w_spec,
              _b_spec],
    out_specs=[_row_spec] * 3, out_shape=[_sds] * 3)

_t_base = pl.pallas_call(
    _base_body, grid=_GRID,
    in_specs=[_row_spec, _w_spec],
    out_specs=_row_spec, out_shape=_sds)

_t_mid = pl.pallas_call(
    _t_mid_body, grid=_GRID,
    in_specs=[_row_spec, _row_spec, _b_spec, _b_spec, _w_spec, _b_spec,
              _w_spec, _b_spec, _row_spec, _w_spec],
    out_specs=[_row_spec] * 4, out_shape=[_sds] * 4)

_t_out = pl.pallas_call(
    _t_out_body, grid=_GRID,
    in_specs=[_row_spec, _row_spec, _b_spec, _b_spec],
    out_specs=_row_spec, out_shape=_sds)


def kernel(node_feat, edge_index_0, edge_attr_0, edge_index_1, edge_attr_1,
           params):
    del edge_attr_1
    nf = jnp.concatenate(
        [node_feat[0], jnp.zeros((NPAD - N, D), f32)], axis=0)
    pad_src = jnp.zeros((EPAD - E,), i32)
    pad_dst = jnp.full((EPAD - E,), NPAD - 1, i32)
    src_all = jnp.concatenate([
        edge_index_0[0], pad_src, edge_index_1[0], pad_src])
    dst_all = jnp.concatenate([
        edge_index_0[1], pad_dst, edge_index_1[1], pad_dst])
    a_rows = jnp.concatenate(
        [edge_attr_0, jnp.ones((E, 1), f32), jnp.zeros((E, D - 5), f32)],
        axis=1)
    a_rows = jnp.concatenate([a_rows, jnp.zeros((EPAD - E, D), f32)],
                             axis=0)
    zrows = jnp.zeros((RPT, D), f32)

    bin_k, s_k, agg_k = _sc_kernels()
    bin_src, bin_dloc, bin_eid, counts = bin_k(src_all, dst_all)
    s_mat = s_k(a_rows, bin_eid, bin_dloc, counts, zrows)

    layers = params["layers"]

    def ew_mat(layer):
        return jnp.concatenate(
            [layer["edge_W"][0], layer["edge_b"][0].reshape(1, D),
             jnp.zeros((D - 5, D), f32)], axis=0)

    l0 = layers[0]
    h, m0, m1 = _t_in(
        nf, params["input_W"], params["input_b"].reshape(1, D),
        l0["node_W"][0], l0["node_b"][0].reshape(1, D),
        l0["node_W"][1], l0["node_b"][1].reshape(1, D))
    base = _t_base(s_mat, ew_mat(l0))

    for li in range(NL):
        agg = agg_k(m0, m1, base, bin_src, bin_dloc, counts)
        lg = layers[li]["gamma"].reshape(1, D)
        lb = layers[li]["beta"].reshape(1, D)
        if li < NL - 1:
            nxt = layers[li + 1]
            h, m0, m1, base = _t_mid(
                h, agg, lg, lb,
                nxt["node_W"][0], nxt["node_b"][0].reshape(1, D),
                nxt["node_W"][1], nxt["node_b"][1].reshape(1, D),
                s_mat, ew_mat(nxt))
        else:
            h = _t_out(h, agg, lg, lb)

    return h[:N].reshape(1, N, D)


# speedup vs baseline: 4.1522x; 1.0449x over previous
import functools

import jax
import jax.numpy as jnp
from jax import lax
from jax.experimental import pallas as pl
from jax.experimental.pallas import tpu as pltpu
from jax.experimental.pallas import tpu_sc as plsc

N = 50000
D = 128
E = 300000
R = 2
NL = 2

NC, NS, LANES = 2, 16, 16
NW = NC * NS

NCHUNK = 4
CHUNK = 12544
NPAD = NCHUNK * CHUNK
CPSC = NCHUNK // NC
DUMP = CHUNK
RPT = CHUNK // NS

EPW = 9376
EPAD = EPW * NW
BATCH = 128
NBAT = 80
CAP = NBAT * BATCH
NPIPE = 1

TBLK = 512

f32 = jnp.float32
i32 = jnp.int32


@functools.lru_cache(maxsize=1)
def _sc_kernels():
    mesh = plsc.VectorSubcoreMesh(core_axis_name="c", subcore_axis_name="s",
                                  num_cores=NC, num_subcores=NS)
    cparams = pltpu.CompilerParams(needs_layout_passes=False)
    bin_k = functools.partial(
        pl.kernel,
        out_type=[
            jax.ShapeDtypeStruct((R * NCHUNK * NW * CAP,), i32),
            jax.ShapeDtypeStruct((R * NCHUNK * NW * CAP,), i32),
            jax.ShapeDtypeStruct((R * NCHUNK * NW * CAP,), i32),
            jax.ShapeDtypeStruct((R * NW * NCHUNK * LANES,), i32),
        ],
        mesh=mesh,
        compiler_params=cparams,
        scratch_types=[
            pltpu.VMEM((EPW,), i32),
            pltpu.VMEM((EPW,), i32),
            pltpu.VMEM((CAP,), i32),
            pltpu.VMEM((CAP,), i32),
            pltpu.VMEM((CAP,), i32),
            pltpu.VMEM((NCHUNK * LANES,), i32),
        ],
    )(_bin_body)
    s_k = functools.partial(
        pl.kernel,
        out_type=jax.ShapeDtypeStruct((NPAD, D), f32),
        mesh=mesh,
        compiler_params=cparams,
        scratch_types=[
            [pltpu.VMEM((BATCH,), i32)] * 4,
            pltpu.VMEM((BATCH, D), f32),
            pltpu.VMEM((NC * NCHUNK * LANES,), i32),
            pltpu.VMEM_SHARED((CHUNK + 1, D), f32),
            [pltpu.SemaphoreType.DMA] * 3,
        ],
    )(_s_body)
    agg_k = functools.partial(
        pl.kernel,
        out_type=jax.ShapeDtypeStruct((NPAD, D), f32),
        mesh=mesh,
        compiler_params=cparams,
        scratch_types=[
            [pltpu.VMEM((BATCH,), i32)] * 4,
            pltpu.VMEM((BATCH, D), f32),
            pltpu.VMEM((R * NC * NCHUNK * LANES,), i32),
            pltpu.VMEM_SHARED((CHUNK + 1, D), f32),
            [pltpu.SemaphoreType.DMA] * 3,
        ],
    )(_agg_body)
    return bin_k, s_k, agg_k


def _bin_body(src_hbm, dst_hbm, bin_src, bin_dloc, bin_eid, counts,
              st_s, st_d, seg_s, seg_dl, seg_e, cnt_st):
    cid_ax = lax.axis_index("c")
    sid_ax = lax.axis_index("s")
    wid = sid_ax * NC + cid_ax
    lanes = lax.iota(i32, LANES)
    zeros16 = jnp.zeros((LANES,), i32)
    dump16 = jnp.full((LANES,), DUMP, i32)
    epad16 = jnp.full((LANES,), E, i32)
    ebase = wid * EPW

    for r in range(R):
        pltpu.sync_copy(src_hbm.at[pl.ds(r * EPAD + ebase, EPW)], st_s)
        pltpu.sync_copy(dst_hbm.at[pl.ds(r * EPAD + ebase, EPW)], st_d)
        for c in range(NCHUNK):
            lo, hi = c * CHUNK, (c + 1) * CHUNK

            def body(i, off):
                s = st_s[pl.ds(i * LANES, LANES)]
                d = st_d[pl.ds(i * LANES, LANES)]
                eid = (ebase + i * LANES) + lanes
                m = (d >= lo) & (d < hi)
                plsc.store_compressed(seg_s.at[pl.ds(off, LANES)], s,
                                      mask=m)
                plsc.store_compressed(seg_dl.at[pl.ds(off, LANES)], d - lo,
                                      mask=m)
                plsc.store_compressed(seg_e.at[pl.ds(off, LANES)], eid,
                                      mask=m)
                return off + plsc.all_reduce_population_count(m)[0]

            off = lax.fori_loop(0, EPW // LANES, body, jnp.int32(0))
            for k in range(NPIPE * BATCH // LANES):
                seg_s[pl.ds(off + k * LANES, LANES)] = zeros16
                seg_dl[pl.ds(off + k * LANES, LANES)] = dump16
                seg_e[pl.ds(off + k * LANES, LANES)] = epad16
            cnt_st[pl.ds(c * LANES, LANES)] = jnp.full((LANES,), off, i32)
            row = (r * NCHUNK + c) * NW + wid
            pltpu.sync_copy(seg_s, bin_src.at[pl.ds(row * CAP, CAP)])
            pltpu.sync_copy(seg_dl, bin_dloc.at[pl.ds(row * CAP, CAP)])
            pltpu.sync_copy(seg_e, bin_eid.at[pl.ds(row * CAP, CAP)])
        pltpu.sync_copy(
            cnt_st,
            counts.at[pl.ds((r * NW + wid) * NCHUNK * LANES,
                            NCHUNK * LANES)])


def _seg_loop(gat_hbm, sca_hbm, base_off, n, table, rows, slots, acc,
              sems):
    s0, d0, s1, d1 = slots
    sem_i0, sem_i1, sem_g = sems
    nb = (n + BATCH - 1) // BATCH
    ng = (nb + 1) // 2

    def drain(sem, da, db):
        pltpu.make_async_copy(gat_hbm.at[pl.ds(0, BATCH)], da, sem).wait()
        pltpu.make_async_copy(gat_hbm.at[pl.ds(0, BATCH)], db, sem).wait()

    @pl.when(nb > 0)
    def _():
        pltpu.async_copy(gat_hbm.at[pl.ds(base_off, BATCH)], s0, sem_i0)
        pltpu.async_copy(sca_hbm.at[pl.ds(base_off, BATCH)], d0, sem_i0)

    def body(g, _):
        off_a = base_off + 2 * g * BATCH
        drain(sem_i0, s0, d0)
        cp = pltpu.async_copy(table.at[s0], rows, sem_g)
        pltpu.async_copy(gat_hbm.at[pl.ds(off_a + BATCH, BATCH)], s1,
                         sem_i1)
        pltpu.async_copy(sca_hbm.at[pl.ds(off_a + BATCH, BATCH)], d1,
                         sem_i1)
        cp.wait()
        pltpu.sync_copy(rows, acc.at[d0], add=True)
        drain(sem_i1, s1, d1)

        @pl.when(2 * g + 1 < nb)
        def _():
            cp2 = pltpu.async_copy(table.at[s1], rows, sem_g)
            cp2.wait()
            pltpu.sync_copy(rows, acc.at[d1], add=True)

        pltpu.async_copy(gat_hbm.at[pl.ds(off_a + 2 * BATCH, BATCH)], s0,
                         sem_i0)
        pltpu.async_copy(sca_hbm.at[pl.ds(off_a + 2 * BATCH, BATCH)], d0,
                         sem_i0)
        return 0

    lax.fori_loop(0, ng, body, 0)

    @pl.when(nb > 0)
    def _():
        drain(sem_i0, s0, d0)


def _s_body(a_hbm, bin_eid, bin_dloc, counts, zrows,
            s_out, slots, arows, cnt, acc, sems):
    cid_ax = lax.axis_index("c")
    sid_ax = lax.axis_index("s")
    pltpu.sync_copy(
        counts.at[pl.ds(2 * sid_ax * NCHUNK * LANES, 2 * NCHUNK * LANES)],
        cnt)

    for cc in range(CPSC):
        chunk = cid_ax + NC * cc
        pltpu.sync_copy(zrows, acc.at[pl.ds(sid_ax * RPT, RPT)])
        plsc.subcore_barrier()
        for sl in range(2):
            seg = 2 * sid_ax + sl
            n = cnt[pl.ds((sl * NCHUNK + chunk) * LANES, LANES)][0]
            base_off = (chunk * NW + seg) * CAP
            _seg_loop(bin_eid, bin_dloc, base_off, n, a_hbm, arows, slots,
                      acc, sems)
        plsc.subcore_barrier()
        pltpu.sync_copy(
            acc.at[pl.ds(sid_ax * RPT, RPT)],
            s_out.at[pl.ds(chunk * CHUNK + sid_ax * RPT, RPT)])
        plsc.subcore_barrier()


def _agg_body(m0_hbm, m1_hbm, base_hbm, bin_src, bin_dloc, counts,
              agg, slots, rows, cnt, acc, sems):
    cid_ax = lax.axis_index("c")
    sid_ax = lax.axis_index("s")
    for r in range(R):
        pltpu.sync_copy(
            counts.at[pl.ds((r * NW + 2 * sid_ax) * NCHUNK * LANES,
                            2 * NCHUNK * LANES)],
            cnt.at[pl.ds(r * 2 * NCHUNK * LANES, 2 * NCHUNK * LANES)])

    for cc in range(CPSC):
        chunk = cid_ax + NC * cc
        rowbase = chunk * CHUNK + sid_ax * RPT
        pltpu.sync_copy(base_hbm.at[pl.ds(rowbase, RPT)],
                        acc.at[pl.ds(sid_ax * RPT, RPT)])
        plsc.subcore_barrier()
        for r in range(R):
            m_hbm = m0_hbm if r == 0 else m1_hbm
            for sl in range(2):
                seg = 2 * sid_ax + sl
                n = cnt[pl.ds(((r * 2 + sl) * NCHUNK + chunk) * LANES,
                              LANES)][0]
                base_off = ((r * NCHUNK + chunk) * NW + seg) * CAP
                _seg_loop(bin_src, bin_dloc, base_off, n, m_hbm, rows,
                          slots, acc, sems)
        plsc.subcore_barrier()
        pltpu.sync_copy(acc.at[pl.ds(sid_ax * RPT, RPT)],
                        agg.at[pl.ds(rowbase, RPT)])
        plsc.subcore_barrier()


def _ln(x, g, b):
    m = jnp.mean(x, axis=-1, keepdims=True)
    xc = x - m
    v = jnp.mean(xc * xc, axis=-1, keepdims=True)
    return g * xc * lax.rsqrt(v + 1e-5) + b


def _mats_body(h, s, w0, b0, w1, b1, ew, m0_ref, m1_ref, base_ref):
    m0_ref[...] = jnp.dot(h, w0, preferred_element_type=f32) + b0
    m1_ref[...] = jnp.dot(h, w1, preferred_element_type=f32) + b1
    base_ref[...] = jnp.dot(s, ew, preferred_element_type=f32)


def _t_in_body(nf_ref, iw_ref, ib_ref, w0_ref, b0_ref, w1_ref, b1_ref,
               h_ref, m0_ref, m1_ref):
    h = jnp.dot(nf_ref[...], iw_ref[...], preferred_element_type=f32) \
        + ib_ref[...]
    h_ref[...] = h
    m0_ref[...] = jnp.dot(h, w0_ref[...], preferred_element_type=f32) \
        + b0_ref[...]
    m1_ref[...] = jnp.dot(h, w1_ref[...], preferred_element_type=f32) \
        + b1_ref[...]


def _base_body(s_ref, ew_ref, base_ref):
    base_ref[...] = jnp.dot(s_ref[...], ew_ref[...],
                            preferred_element_type=f32)


def _t_mid_body(hp_ref, ag_ref, g_ref, be_ref, w0_ref, b0_ref, w1_ref,
                b1_ref, s_ref, ew_ref, h_ref, m0_ref, m1_ref, base_ref):
    x = hp_ref[...] + jnp.maximum(ag_ref[...], 0.0)
    h = _ln(x, g_ref[...], be_ref[...])
    h_ref[...] = h
    _mats_body(h, s_ref[...], w0_ref[...], b0_ref[...], w1_ref[...],
               b1_ref[...], ew_ref[...], m0_ref, m1_ref, base_ref)


def _t_out_body(hp_ref, ag_ref, g_ref, be_ref, h_ref):
    x = hp_ref[...] + jnp.maximum(ag_ref[...], 0.0)
    h_ref[...] = _ln(x, g_ref[...], be_ref[...])


_row_spec = pl.BlockSpec((TBLK, D), lambda i: (i, 0))
_w_spec = pl.BlockSpec((D, D), lambda i: (0, 0))
_b_spec = pl.BlockSpec((1, D), lambda i: (0, 0))
_GRID = (NPAD // TBLK,)
_sds = jax.ShapeDtypeStruct((NPAD, D), f32)

_t_in = pl.pallas_call(
    _t_in_body, grid=_GRID,
    in_specs=[_row_spec, _w_spec, _b_spec, _w_spec, _b_spec, _w_spec,
              _b_spec],
    out_specs=[_row_spec] * 3, out_shape=[_sds] * 3)

_t_base = pl.pallas_call(
    _base_body, grid=_GRID,
    in_specs=[_row_spec, _w_spec],
    out_specs=_row_spec, out_shape=_sds)

_t_mid = pl.pallas_call(
    _t_mid_body, grid=_GRID,
    in_specs=[_row_spec, _row_spec, _b_spec, _b_spec, _w_spec, _b_spec,
              _w_spec, _b_spec, _row_spec, _w_spec],
    out_specs=[_row_spec] * 4, out_shape=[_sds] * 4)

_t_out = pl.pallas_call(
    _t_out_body, grid=_GRID,
    in_specs=[_row_spec, _row_spec, _b_spec, _b_spec],
    out_specs=_row_spec, out_shape=_sds)


def kernel(node_feat, edge_index_0, edge_attr_0, edge_index_1, edge_attr_1,
           params):
    del edge_attr_1
    nf = jnp.concatenate(
        [node_feat[0], jnp.zeros((NPAD - N, D), f32)], axis=0)
    pad_src = jnp.zeros((EPAD - E,), i32)
    pad_dst = jnp.full((EPAD - E,), NPAD - 1, i32)
    src_all = jnp.concatenate([
        edge_index_0[0], pad_src, edge_index_1[0], pad_src])
    dst_all = jnp.concatenate([
        edge_index_0[1], pad_dst, edge_index_1[1], pad_dst])
    a_rows = jnp.concatenate(
        [edge_attr_0, jnp.ones((E, 1), f32), jnp.zeros((E, D - 5), f32)],
        axis=1)
    a_rows = jnp.concatenate([a_rows, jnp.zeros((EPAD - E, D), f32)],
                             axis=0)
    zrows = jnp.zeros((RPT, D), f32)

    bin_k, s_k, agg_k = _sc_kernels()
    bin_src, bin_dloc, bin_eid, counts = bin_k(src_all, dst_all)
    s_mat = s_k(a_rows, bin_eid, bin_dloc, counts, zrows)

    layers = params["layers"]

    def ew_mat(layer):
        return jnp.concatenate(
            [layer["edge_W"][0], layer["edge_b"][0].reshape(1, D),
             jnp.zeros((D - 5, D), f32)], axis=0)

    l0 = layers[0]
    h, m0, m1 = _t_in(
        nf, params["input_W"], params["input_b"].reshape(1, D),
        l0["node_W"][0], l0["node_b"][0].reshape(1, D),
        l0["node_W"][1], l0["node_b"][1].reshape(1, D))
    base = _t_base(s_mat, ew_mat(l0))

    for li in range(NL):
        agg = agg_k(m0, m1, base, bin_src, bin_dloc, counts)
        lg = layers[li]["gamma"].reshape(1, D)
        lb = layers[li]["beta"].reshape(1, D)
        if li < NL - 1:
            nxt = layers[li + 1]
            h, m0, m1, base = _t_mid(
                h, agg, lg, lb,
                nxt["node_W"][0], nxt["node_b"][0].reshape(1, D),
                nxt["node_W"][1], nxt["node_b"][1].reshape(1, D),
                s_mat, ew_mat(nxt))
        else:
            h = _t_out(h, agg, lg, lb)

    return h[:N].reshape(1, N, D)


# speedup vs baseline: 4.6545x; 1.1210x over previous
import functools

import jax
import jax.numpy as jnp
from jax import lax
from jax.experimental import pallas as pl
from jax.experimental.pallas import tpu as pltpu
from jax.experimental.pallas import tpu_sc as plsc

N = 50000
D = 128
E = 300000
R = 2
NL = 2

NC, NS, LANES = 2, 16, 16
NW = NC * NS

NCHUNK = 4
CHUNK = 12544
NPAD = NCHUNK * CHUNK
CPSC = NCHUNK // NC
DUMP = CHUNK
RPT = CHUNK // NS

EPW = 9376
EPAD = EPW * NW
BATCH = 112
CAP = 86 * BATCH
NPIPE = 1

TBLK = 512

f32 = jnp.float32
i32 = jnp.int32


@functools.lru_cache(maxsize=1)
def _sc_kernels():
    mesh = plsc.VectorSubcoreMesh(core_axis_name="c", subcore_axis_name="s",
                                  num_cores=NC, num_subcores=NS)
    cparams = pltpu.CompilerParams(needs_layout_passes=False)
    bin_k = functools.partial(
        pl.kernel,
        out_type=[
            jax.ShapeDtypeStruct((R * NCHUNK * NW * CAP,), i32),
            jax.ShapeDtypeStruct((R * NCHUNK * NW * CAP,), i32),
            jax.ShapeDtypeStruct((R * NCHUNK * NW * CAP,), i32),
            jax.ShapeDtypeStruct((R * NW * NCHUNK * LANES,), i32),
        ],
        mesh=mesh,
        compiler_params=cparams,
        scratch_types=[
            pltpu.VMEM((EPW,), i32),
            pltpu.VMEM((EPW,), i32),
            pltpu.VMEM((CAP,), i32),
            pltpu.VMEM((CAP,), i32),
            pltpu.VMEM((CAP,), i32),
            pltpu.VMEM((NCHUNK * LANES,), i32),
        ],
    )(_bin_body)
    s_k = functools.partial(
        pl.kernel,
        out_type=jax.ShapeDtypeStruct((NPAD, D), f32),
        mesh=mesh,
        compiler_params=cparams,
        scratch_types=[
            [pltpu.VMEM((BATCH,), i32)] * 4,
            pltpu.VMEM((2, BATCH, D), f32),
            pltpu.VMEM((NC * NCHUNK * LANES,), i32),
            pltpu.VMEM_SHARED((CHUNK + 1, D), f32),
            [pltpu.SemaphoreType.DMA] * 4,
        ],
    )(_s_body)
    agg_k = functools.partial(
        pl.kernel,
        out_type=jax.ShapeDtypeStruct((NPAD, D), f32),
        mesh=mesh,
        compiler_params=cparams,
        scratch_types=[
            [pltpu.VMEM((BATCH,), i32)] * 4,
            pltpu.VMEM((2, BATCH, D), f32),
            pltpu.VMEM((R * NC * NCHUNK * LANES,), i32),
            pltpu.VMEM_SHARED((CHUNK + 1, D), f32),
            [pltpu.SemaphoreType.DMA] * 4,
        ],
    )(_agg_body)
    return bin_k, s_k, agg_k


def _bin_body(src_hbm, dst_hbm, bin_src, bin_dloc, bin_eid, counts,
              st_s, st_d, seg_s, seg_dl, seg_e, cnt_st):
    cid_ax = lax.axis_index("c")
    sid_ax = lax.axis_index("s")
    wid = sid_ax * NC + cid_ax
    lanes = lax.iota(i32, LANES)
    zeros16 = jnp.zeros((LANES,), i32)
    dump16 = jnp.full((LANES,), DUMP, i32)
    epad16 = jnp.full((LANES,), E, i32)
    ebase = wid * EPW

    for r in range(R):
        pltpu.sync_copy(src_hbm.at[pl.ds(r * EPAD + ebase, EPW)], st_s)
        pltpu.sync_copy(dst_hbm.at[pl.ds(r * EPAD + ebase, EPW)], st_d)
        for c in range(NCHUNK):
            lo, hi = c * CHUNK, (c + 1) * CHUNK

            def body(i, off):
                s = st_s[pl.ds(i * LANES, LANES)]
                d = st_d[pl.ds(i * LANES, LANES)]
                eid = (ebase + i * LANES) + lanes
                m = (d >= lo) & (d < hi)
                plsc.store_compressed(seg_s.at[pl.ds(off, LANES)], s,
                                      mask=m)
                plsc.store_compressed(seg_dl.at[pl.ds(off, LANES)], d - lo,
                                      mask=m)
                plsc.store_compressed(seg_e.at[pl.ds(off, LANES)], eid,
                                      mask=m)
                return off + plsc.all_reduce_population_count(m)[0]

            off = lax.fori_loop(0, EPW // LANES, body, jnp.int32(0))
            for k in range(NPIPE * BATCH // LANES):
                seg_s[pl.ds(off + k * LANES, LANES)] = zeros16
                seg_dl[pl.ds(off + k * LANES, LANES)] = dump16
                seg_e[pl.ds(off + k * LANES, LANES)] = epad16
            cnt_st[pl.ds(c * LANES, LANES)] = jnp.full((LANES,), off, i32)
            row = (r * NCHUNK + c) * NW + wid
            pltpu.sync_copy(seg_s, bin_src.at[pl.ds(row * CAP, CAP)])
            pltpu.sync_copy(seg_dl, bin_dloc.at[pl.ds(row * CAP, CAP)])
            pltpu.sync_copy(seg_e, bin_eid.at[pl.ds(row * CAP, CAP)])
        pltpu.sync_copy(
            cnt_st,
            counts.at[pl.ds((r * NW + wid) * NCHUNK * LANES,
                            NCHUNK * LANES)])


def _seg_loop(gat_hbm, sca_hbm, base_off, n, table, rows, slots, acc,
              sems):
    s0, d0, s1, d1 = slots
    sem_i0, sem_i1, sem_g0, sem_g1 = sems
    nb = (n + BATCH - 1) // BATCH
    ng = (nb + 1) // 2

    def drain(sem, da, db):
        pltpu.make_async_copy(gat_hbm.at[pl.ds(0, BATCH)], da, sem).wait()
        pltpu.make_async_copy(gat_hbm.at[pl.ds(0, BATCH)], db, sem).wait()

    @pl.when(nb > 0)
    def _():
        pltpu.async_copy(gat_hbm.at[pl.ds(base_off, BATCH)], s0, sem_i0)
        pltpu.async_copy(sca_hbm.at[pl.ds(base_off, BATCH)], d0, sem_i0)

    def body(g, _):
        off_a = base_off + 2 * g * BATCH
        drain(sem_i0, s0, d0)
        cp = pltpu.async_copy(table.at[s0], rows.at[0], sem_g0)
        pltpu.async_copy(gat_hbm.at[pl.ds(off_a + BATCH, BATCH)], s1,
                         sem_i1)
        pltpu.async_copy(sca_hbm.at[pl.ds(off_a + BATCH, BATCH)], d1,
                         sem_i1)
        cp.wait()
        drain(sem_i1, s1, d1)

        @pl.when(2 * g + 1 < nb)
        def _():
            cp2 = pltpu.async_copy(table.at[s1], rows.at[1], sem_g1)
            pltpu.sync_copy(rows.at[0], acc.at[d0], add=True)
            cp2.wait()
            pltpu.sync_copy(rows.at[1], acc.at[d1], add=True)

        @pl.when(2 * g + 1 >= nb)
        def _():
            pltpu.sync_copy(rows.at[0], acc.at[d0], add=True)

        pltpu.async_copy(gat_hbm.at[pl.ds(off_a + 2 * BATCH, BATCH)], s0,
                         sem_i0)
        pltpu.async_copy(sca_hbm.at[pl.ds(off_a + 2 * BATCH, BATCH)], d0,
                         sem_i0)
        return 0

    lax.fori_loop(0, ng, body, 0)

    @pl.when(nb > 0)
    def _():
        drain(sem_i0, s0, d0)


def _s_body(a_hbm, bin_eid, bin_dloc, counts, zrows,
            s_out, slots, arows, cnt, acc, sems):
    cid_ax = lax.axis_index("c")
    sid_ax = lax.axis_index("s")
    pltpu.sync_copy(
        counts.at[pl.ds(2 * sid_ax * NCHUNK * LANES, 2 * NCHUNK * LANES)],
        cnt)

    for cc in range(CPSC):
        chunk = cid_ax + NC * cc
        pltpu.sync_copy(zrows, acc.at[pl.ds(sid_ax * RPT, RPT)])
        plsc.subcore_barrier()
        for sl in range(2):
            seg = 2 * sid_ax + sl
            n = cnt[pl.ds((sl * NCHUNK + chunk) * LANES, LANES)][0]
            base_off = (chunk * NW + seg) * CAP
            _seg_loop(bin_eid, bin_dloc, base_off, n, a_hbm, arows, slots,
                      acc, sems)
        plsc.subcore_barrier()
        pltpu.sync_copy(
            acc.at[pl.ds(sid_ax * RPT, RPT)],
            s_out.at[pl.ds(chunk * CHUNK + sid_ax * RPT, RPT)])
        plsc.subcore_barrier()


def _agg_body(m0_hbm, m1_hbm, base_hbm, bin_src, bin_dloc, counts,
              agg, slots, rows, cnt, acc, sems):
    cid_ax = lax.axis_index("c")
    sid_ax = lax.axis_index("s")
    for r in range(R):
        pltpu.sync_copy(
            counts.at[pl.ds((r * NW + 2 * sid_ax) * NCHUNK * LANES,
                            2 * NCHUNK * LANES)],
            cnt.at[pl.ds(r * 2 * NCHUNK * LANES, 2 * NCHUNK * LANES)])

    for cc in range(CPSC):
        chunk = cid_ax + NC * cc
        rowbase = chunk * CHUNK + sid_ax * RPT
        pltpu.sync_copy(base_hbm.at[pl.ds(rowbase, RPT)],
                        acc.at[pl.ds(sid_ax * RPT, RPT)])
        plsc.subcore_barrier()
        for r in range(R):
            m_hbm = m0_hbm if r == 0 else m1_hbm
            for sl in range(2):
                seg = 2 * sid_ax + sl
                n = cnt[pl.ds(((r * 2 + sl) * NCHUNK + chunk) * LANES,
                              LANES)][0]
                base_off = ((r * NCHUNK + chunk) * NW + seg) * CAP
                _seg_loop(bin_src, bin_dloc, base_off, n, m_hbm, rows,
                          slots, acc, sems)
        plsc.subcore_barrier()
        pltpu.sync_copy(acc.at[pl.ds(sid_ax * RPT, RPT)],
                        agg.at[pl.ds(rowbase, RPT)])
        plsc.subcore_barrier()


def _ln(x, g, b):
    m = jnp.mean(x, axis=-1, keepdims=True)
    xc = x - m
    v = jnp.mean(xc * xc, axis=-1, keepdims=True)
    return g * xc * lax.rsqrt(v + 1e-5) + b


def _mats_body(h, s, w0, b0, w1, b1, ew, m0_ref, m1_ref, base_ref):
    m0_ref[...] = jnp.dot(h, w0, preferred_element_type=f32) + b0
    m1_ref[...] = jnp.dot(h, w1, preferred_element_type=f32) + b1
    base_ref[...] = jnp.dot(s, ew, preferred_element_type=f32)


def _t_in_body(nf_ref, iw_ref, ib_ref, w0_ref, b0_ref, w1_ref, b1_ref,
               h_ref, m0_ref, m1_ref):
    h = jnp.dot(nf_ref[...], iw_ref[...], preferred_element_type=f32) \
        + ib_ref[...]
    h_ref[...] = h
    m0_ref[...] = jnp.dot(h, w0_ref[...], preferred_element_type=f32) \
        + b0_ref[...]
    m1_ref[...] = jnp.dot(h, w1_ref[...], preferred_element_type=f32) \
        + b1_ref[...]


def _base_body(s_ref, ew_ref, base_ref):
    base_ref[...] = jnp.dot(s_ref[...], ew_ref[...],
                            preferred_element_type=f32)


def _t_mid_body(hp_ref, ag_ref, g_ref, be_ref, w0_ref, b0_ref, w1_ref,
                b1_ref, s_ref, ew_ref, h_ref, m0_ref, m1_ref, base_ref):
    x = hp_ref[...] + jnp.maximum(ag_ref[...], 0.0)
    h = _ln(x, g_ref[...], be_ref[...])
    h_ref[...] = h
    _mats_body(h, s_ref[...], w0_ref[...], b0_ref[...], w1_ref[...],
               b1_ref[...], ew_ref[...], m0_ref, m1_ref, base_ref)


def _t_out_body(hp_ref, ag_ref, g_ref, be_ref, h_ref):
    x = hp_ref[...] + jnp.maximum(ag_ref[...], 0.0)
    h_ref[...] = _ln(x, g_ref[...], be_ref[...])


_row_spec = pl.BlockSpec((TBLK, D), lambda i: (i, 0))
_w_spec = pl.BlockSpec((D, D), lambda i: (0, 0))
_b_spec = pl.BlockSpec((1, D), lambda i: (0, 0))
_GRID = (NPAD // TBLK,)
_sds = jax.ShapeDtypeStruct((NPAD, D), f32)

_t_in = pl.pallas_call(
    _t_in_body, grid=_GRID,
    in_specs=[_row_spec, _w_spec, _b_spec, _w_spec, _b_spec, _w_spec,
              _b_spec],
    out_specs=[_row_spec] * 3, out_shape=[_sds] * 3)

_t_base = pl.pallas_call(
    _base_body, grid=_GRID,
    in_specs=[_row_spec, _w_spec],
    out_specs=_row_spec, out_shape=_sds)

_t_mid = pl.pallas_call(
    _t_mid_body, grid=_GRID,
    in_specs=[_row_spec, _row_spec, _b_spec, _b_spec, _w_spec, _b_spec,
              _w_spec, _b_spec, _row_spec, _w_spec],
    out_specs=[_row_spec] * 4, out_shape=[_sds] * 4)

_t_out = pl.pallas_call(
    _t_out_body, grid=_GRID,
    in_specs=[_row_spec, _row_spec, _b_spec, _b_spec],
    out_specs=_row_spec, out_shape=_sds)


def kernel(node_feat, edge_index_0, edge_attr_0, edge_index_1, edge_attr_1,
           params):
    del edge_attr_1
    nf = jnp.concatenate(
        [node_feat[0], jnp.zeros((NPAD - N, D), f32)], axis=0)
    pad_src = jnp.zeros((EPAD - E,), i32)
    pad_dst = jnp.full((EPAD - E,), NPAD - 1, i32)
    src_all = jnp.concatenate([
        edge_index_0[0], pad_src, edge_index_1[0], pad_src])
    dst_all = jnp.concatenate([
        edge_index_0[1], pad_dst, edge_index_1[1], pad_dst])
    a_rows = jnp.concatenate(
        [edge_attr_0, jnp.ones((E, 1), f32), jnp.zeros((E, D - 5), f32)],
        axis=1)
    a_rows = jnp.concatenate([a_rows, jnp.zeros((EPAD - E, D), f32)],
                             axis=0)
    zrows = jnp.zeros((RPT, D), f32)

    bin_k, s_k, agg_k = _sc_kernels()
    bin_src, bin_dloc, bin_eid, counts = bin_k(src_all, dst_all)
    s_mat = s_k(a_rows, bin_eid, bin_dloc, counts, zrows)

    layers = params["layers"]

    def ew_mat(layer):
        return jnp.concatenate(
            [layer["edge_W"][0], layer["edge_b"][0].reshape(1, D),
             jnp.zeros((D - 5, D), f32)], axis=0)

    l0 = layers[0]
    h, m0, m1 = _t_in(
        nf, params["input_W"], params["input_b"].reshape(1, D),
        l0["node_W"][0], l0["node_b"][0].reshape(1, D),
        l0["node_W"][1], l0["node_b"][1].reshape(1, D))
    base = _t_base(s_mat, ew_mat(l0))

    for li in range(NL):
        agg = agg_k(m0, m1, base, bin_src, bin_dloc, counts)
        lg = layers[li]["gamma"].reshape(1, D)
        lb = layers[li]["beta"].reshape(1, D)
        if li < NL - 1:
            nxt = layers[li + 1]
            h, m0, m1, base = _t_mid(
                h, agg, lg, lb,
                nxt["node_W"][0], nxt["node_b"][0].reshape(1, D),
                nxt["node_W"][1], nxt["node_b"][1].reshape(1, D),
                s_mat, ew_mat(nxt))
        else:
            h = _t_out(h, agg, lg, lb)

    return h[:N].reshape(1, N, D)


# speedup vs baseline: 4.6844x; 1.0064x over previous
import functools

import jax
import jax.numpy as jnp
from jax import lax
from jax.experimental import pallas as pl
from jax.experimental.pallas import tpu as pltpu
from jax.experimental.pallas import tpu_sc as plsc

N = 50000
D = 128
E = 300000
R = 2
NL = 2

NC, NS, LANES = 2, 16, 16
NW = NC * NS

NCHUNK = 4
CHUNK = 12544
NPAD = NCHUNK * CHUNK
CPSC = NCHUNK // NC
DUMP = CHUNK
RPT = CHUNK // NS

EPW = 9376
EPAD = EPW * NW
BATCH = 112
CAP = 86 * BATCH
NPIPE = 1

TBLK = 512

f32 = jnp.float32
i32 = jnp.int32


@functools.lru_cache(maxsize=1)
def _sc_kernels():
    mesh = plsc.VectorSubcoreMesh(core_axis_name="c", subcore_axis_name="s",
                                  num_cores=NC, num_subcores=NS)
    cparams = pltpu.CompilerParams(needs_layout_passes=False)
    bin_k = functools.partial(
        pl.kernel,
        out_type=[
            jax.ShapeDtypeStruct((R * NCHUNK * NW * CAP,), i32),
            jax.ShapeDtypeStruct((R * NCHUNK * NW * CAP,), i32),
            jax.ShapeDtypeStruct((R * NCHUNK * NW * CAP,), i32),
            jax.ShapeDtypeStruct((R * NW * NCHUNK * LANES,), i32),
        ],
        mesh=mesh,
        compiler_params=cparams,
        scratch_types=[
            pltpu.VMEM((EPW,), i32),
            pltpu.VMEM((EPW,), i32),
            pltpu.VMEM((CAP,), i32),
            pltpu.VMEM((CAP,), i32),
            pltpu.VMEM((CAP,), i32),
            pltpu.VMEM((NCHUNK * LANES,), i32),
        ],
    )(_bin_body)
    s_k = functools.partial(
        pl.kernel,
        out_type=jax.ShapeDtypeStruct((NPAD, D), f32),
        mesh=mesh,
        compiler_params=cparams,
        scratch_types=[
            [pltpu.VMEM((BATCH,), i32)] * 4,
            pltpu.VMEM((2, BATCH, D), f32),
            pltpu.VMEM((NC * NCHUNK * LANES,), i32),
            pltpu.VMEM_SHARED((CHUNK + 1, D), f32),
            [pltpu.SemaphoreType.DMA] * 4,
        ],
    )(_s_body)
    agg_k = functools.partial(
        pl.kernel,
        out_type=jax.ShapeDtypeStruct((NPAD, D), f32),
        mesh=mesh,
        compiler_params=cparams,
        scratch_types=[
            [pltpu.VMEM((BATCH,), i32)] * 4,
            pltpu.VMEM((2, BATCH, D), f32),
            pltpu.VMEM((R * NC * NCHUNK * LANES,), i32),
            pltpu.VMEM_SHARED((CHUNK + 1, D), f32),
            [pltpu.SemaphoreType.DMA] * 4,
        ],
    )(_agg_body)
    return bin_k, s_k, agg_k


def _bin_body(src_hbm, dst_hbm, bin_src, bin_dloc, bin_eid, counts,
              st_s, st_d, seg_s, seg_dl, seg_e, cnt_st):
    cid_ax = lax.axis_index("c")
    sid_ax = lax.axis_index("s")
    wid = sid_ax * NC + cid_ax
    lanes = lax.iota(i32, LANES)
    zeros16 = jnp.zeros((LANES,), i32)
    dump16 = jnp.full((LANES,), DUMP, i32)
    epad16 = jnp.full((LANES,), E, i32)
    ebase = wid * EPW

    for r in range(R):
        pltpu.sync_copy(src_hbm.at[pl.ds(r * EPAD + ebase, EPW)], st_s)
        pltpu.sync_copy(dst_hbm.at[pl.ds(r * EPAD + ebase, EPW)], st_d)
        for c in range(NCHUNK):
            lo, hi = c * CHUNK, (c + 1) * CHUNK

            def body(i, off):
                s = st_s[pl.ds(i * LANES, LANES)]
                d = st_d[pl.ds(i * LANES, LANES)]
                eid = (ebase + i * LANES) + lanes
                m = (d >= lo) & (d < hi)
                plsc.store_compressed(seg_s.at[pl.ds(off, LANES)], s,
                                      mask=m)
                plsc.store_compressed(seg_dl.at[pl.ds(off, LANES)], d - lo,
                                      mask=m)
                plsc.store_compressed(seg_e.at[pl.ds(off, LANES)], eid,
                                      mask=m)
                return off + plsc.all_reduce_population_count(m)[0]

            off = lax.fori_loop(0, EPW // LANES, body, jnp.int32(0))
            for k in range(NPIPE * BATCH // LANES):
                seg_s[pl.ds(off + k * LANES, LANES)] = zeros16
                seg_dl[pl.ds(off + k * LANES, LANES)] = dump16
                seg_e[pl.ds(off + k * LANES, LANES)] = epad16
            cnt_st[pl.ds(c * LANES, LANES)] = jnp.full((LANES,), off, i32)
            row = (r * NCHUNK + c) * NW + wid
            pltpu.sync_copy(seg_s, bin_src.at[pl.ds(row * CAP, CAP)])
            pltpu.sync_copy(seg_dl, bin_dloc.at[pl.ds(row * CAP, CAP)])
            pltpu.sync_copy(seg_e, bin_eid.at[pl.ds(row * CAP, CAP)])
        pltpu.sync_copy(
            cnt_st,
            counts.at[pl.ds((r * NW + wid) * NCHUNK * LANES,
                            NCHUNK * LANES)])


def _seg_loop(gat_hbm, sca_hbm, base_off, n, table, rows, slots, acc,
              sems):
    s0, d0, s1, d1 = slots
    sem_i0, sem_i1, sem_g0, sem_g1 = sems
    nb = (n + BATCH - 1) // BATCH
    ng = (nb + 1) // 2

    def drain(sem, da, db):
        pltpu.make_async_copy(gat_hbm.at[pl.ds(0, BATCH)], da, sem).wait()
        pltpu.make_async_copy(gat_hbm.at[pl.ds(0, BATCH)], db, sem).wait()

    @pl.when(nb > 0)
    def _():
        pltpu.async_copy(gat_hbm.at[pl.ds(base_off, BATCH)], s0, sem_i0)
        pltpu.async_copy(sca_hbm.at[pl.ds(base_off, BATCH)], d0, sem_i0)

    def body(g, _):
        off_a = base_off + 2 * g * BATCH
        drain(sem_i0, s0, d0)
        cp = pltpu.async_copy(table.at[s0], rows.at[0], sem_g0)
        pltpu.async_copy(gat_hbm.at[pl.ds(off_a + BATCH, BATCH)], s1,
                         sem_i1)
        pltpu.async_copy(sca_hbm.at[pl.ds(off_a + BATCH, BATCH)], d1,
                         sem_i1)
        cp.wait()
        drain(sem_i1, s1, d1)

        @pl.when(2 * g + 1 < nb)
        def _():
            cp2 = pltpu.async_copy(table.at[s1], rows.at[1], sem_g1)
            pltpu.sync_copy(rows.at[0], acc.at[d0], add=True)
            cp2.wait()
            pltpu.sync_copy(rows.at[1], acc.at[d1], add=True)

        @pl.when(2 * g + 1 >= nb)
        def _():
            pltpu.sync_copy(rows.at[0], acc.at[d0], add=True)

        pltpu.async_copy(gat_hbm.at[pl.ds(off_a + 2 * BATCH, BATCH)], s0,
                         sem_i0)
        pltpu.async_copy(sca_hbm.at[pl.ds(off_a + 2 * BATCH, BATCH)], d0,
                         sem_i0)
        return 0

    lax.fori_loop(0, ng, body, 0)

    @pl.when(nb > 0)
    def _():
        drain(sem_i0, s0, d0)


def _s_body(a_hbm, bin_eid, bin_dloc, counts, zrows,
            s_out, slots, arows, cnt, acc, sems):
    cid_ax = lax.axis_index("c")
    sid_ax = lax.axis_index("s")
    pltpu.sync_copy(
        counts.at[pl.ds(2 * sid_ax * NCHUNK * LANES, 2 * NCHUNK * LANES)],
        cnt)

    for cc in range(CPSC):
        chunk = cid_ax + NC * cc
        pltpu.sync_copy(zrows, acc.at[pl.ds(sid_ax * RPT, RPT)])
        plsc.subcore_barrier()
        for sl in range(2):
            seg = 2 * sid_ax + sl
            n = cnt[pl.ds((sl * NCHUNK + chunk) * LANES, LANES)][0]
            base_off = (chunk * NW + seg) * CAP
            _seg_loop(bin_eid, bin_dloc, base_off, n, a_hbm, arows, slots,
                      acc, sems)
        plsc.subcore_barrier()
        pltpu.sync_copy(
            acc.at[pl.ds(sid_ax * RPT, RPT)],
            s_out.at[pl.ds(chunk * CHUNK + sid_ax * RPT, RPT)])
        plsc.subcore_barrier()


def _agg_body(m0_hbm, m1_hbm, base_hbm, bin_src, bin_dloc, counts,
              agg, slots, rows, cnt, acc, sems):
    cid_ax = lax.axis_index("c")
    sid_ax = lax.axis_index("s")
    for r in range(R):
        pltpu.sync_copy(
            counts.at[pl.ds((r * NW + 2 * sid_ax) * NCHUNK * LANES,
                            2 * NCHUNK * LANES)],
            cnt.at[pl.ds(r * 2 * NCHUNK * LANES, 2 * NCHUNK * LANES)])

    for cc in range(CPSC):
        chunk = cid_ax + NC * cc
        rowbase = chunk * CHUNK + sid_ax * RPT
        pltpu.sync_copy(base_hbm.at[pl.ds(rowbase, RPT)],
                        acc.at[pl.ds(sid_ax * RPT, RPT)])
        plsc.subcore_barrier()
        for r in range(R):
            m_hbm = m0_hbm if r == 0 else m1_hbm
            for sl in range(2):
                seg = 2 * sid_ax + sl
                n = cnt[pl.ds(((r * 2 + sl) * NCHUNK + chunk) * LANES,
                              LANES)][0]
                base_off = ((r * NCHUNK + chunk) * NW + seg) * CAP
                _seg_loop(bin_src, bin_dloc, base_off, n, m_hbm, rows,
                          slots, acc, sems)
        plsc.subcore_barrier()
        pltpu.sync_copy(acc.at[pl.ds(sid_ax * RPT, RPT)],
                        agg.at[pl.ds(rowbase, RPT)])
        plsc.subcore_barrier()


def _ln(x, g, b):
    m = jnp.mean(x, axis=-1, keepdims=True)
    xc = x - m
    v = jnp.mean(xc * xc, axis=-1, keepdims=True)
    return g * xc * lax.rsqrt(v + 1e-5) + b


def _mats_body(h, s, w0, b0, w1, b1, ew, m0_ref, m1_ref, base_ref):
    m0_ref[...] = jnp.dot(h, w0, preferred_element_type=f32) + b0
    m1_ref[...] = jnp.dot(h, w1, preferred_element_type=f32) + b1
    base_ref[...] = jnp.dot(s, ew, preferred_element_type=f32)


def _t_in_body(nf_ref, iw_ref, ib_ref, w0_ref, b0_ref, w1_ref, b1_ref,
               h_ref, m0_ref, m1_ref):
    h = jnp.dot(nf_ref[...], iw_ref[...], preferred_element_type=f32) \
        + ib_ref[...]
    h_ref[...] = h
    m0_ref[...] = jnp.dot(h, w0_ref[...], preferred_element_type=f32) \
        + b0_ref[...]
    m1_ref[...] = jnp.dot(h, w1_ref[...], preferred_element_type=f32) \
        + b1_ref[...]


def _base_body(s_ref, ew_ref, base_ref):
    base_ref[...] = jnp.dot(s_ref[...], ew_ref[...],
                            preferred_element_type=f32)


def _t_mid_body(hp_ref, ag_ref, g_ref, be_ref, w0_ref, b0_ref, w1_ref,
                b1_ref, s_ref, ew_ref, h_ref, m0_ref, m1_ref, base_ref):
    x = hp_ref[...] + jnp.maximum(ag_ref[...], 0.0)
    h = _ln(x, g_ref[...], be_ref[...])
    h_ref[...] = h
    _mats_body(h, s_ref[...], w0_ref[...], b0_ref[...], w1_ref[...],
               b1_ref[...], ew_ref[...], m0_ref, m1_ref, base_ref)


def _t_out_body(hp_ref, ag_ref, g_ref, be_ref, h_ref):
    x = hp_ref[...] + jnp.maximum(ag_ref[...], 0.0)
    h_ref[...] = _ln(x, g_ref[...], be_ref[...])


_row_spec = pl.BlockSpec((TBLK, D), lambda i: (i, 0))
_w_spec = pl.BlockSpec((D, D), lambda i: (0, 0))
_b_spec = pl.BlockSpec((1, D), lambda i: (0, 0))
_GRID = (NPAD // TBLK,)
_sds = jax.ShapeDtypeStruct((NPAD, D), f32)

_t_in = pl.pallas_call(
    _t_in_body, grid=_GRID,
    in_specs=[_row_spec, _w_spec, _b_spec, _w_spec, _b_spec, _w_spec,
              _b_spec],
    out_specs=[_row_spec] * 3, out_shape=[_sds] * 3)

_t_base = pl.pallas_call(
    _base_body, grid=_GRID,
    in_specs=[_row_spec, _w_spec],
    out_specs=_row_spec, out_shape=_sds)

_t_mid = pl.pallas_call(
    _t_mid_body, grid=_GRID,
    in_specs=[_row_spec, _row_spec, _b_spec, _b_spec, _w_spec, _b_spec,
              _w_spec, _b_spec, _row_spec, _w_spec],
    out_specs=[_row_spec] * 4, out_shape=[_sds] * 4)

_t_out = pl.pallas_call(
    _t_out_body, grid=_GRID,
    in_specs=[_row_spec, _row_spec, _b_spec, _b_spec],
    out_specs=_row_spec, out_shape=jax.ShapeDtypeStruct((N, D), f32))


def kernel(node_feat, edge_index_0, edge_attr_0, edge_index_1, edge_attr_1,
           params):
    del edge_attr_1
    nf = jnp.concatenate(
        [node_feat[0], jnp.zeros((NPAD - N, D), f32)], axis=0)
    pad_src = jnp.zeros((EPAD - E,), i32)
    pad_dst = jnp.full((EPAD - E,), NPAD - 1, i32)
    src_all = jnp.concatenate([
        edge_index_0[0], pad_src, edge_index_1[0], pad_src])
    dst_all = jnp.concatenate([
        edge_index_0[1], pad_dst, edge_index_1[1], pad_dst])
    a_rows = jnp.concatenate(
        [edge_attr_0, jnp.ones((E, 1), f32), jnp.zeros((E, D - 5), f32)],
        axis=1)
    a_rows = jnp.concatenate([a_rows, jnp.zeros((EPAD - E, D), f32)],
                             axis=0)
    zrows = jnp.zeros((RPT, D), f32)

    bin_k, s_k, agg_k = _sc_kernels()
    bin_src, bin_dloc, bin_eid, counts = bin_k(src_all, dst_all)
    s_mat = s_k(a_rows, bin_eid, bin_dloc, counts, zrows)

    layers = params["layers"]

    def ew_mat(layer):
        return jnp.concatenate(
            [layer["edge_W"][0], layer["edge_b"][0].reshape(1, D),
             jnp.zeros((D - 5, D), f32)], axis=0)

    l0 = layers[0]
    h, m0, m1 = _t_in(
        nf, params["input_W"], params["input_b"].reshape(1, D),
        l0["node_W"][0], l0["node_b"][0].reshape(1, D),
        l0["node_W"][1], l0["node_b"][1].reshape(1, D))
    base = _t_base(s_mat, ew_mat(l0))

    for li in range(NL):
        agg = agg_k(m0, m1, base, bin_src, bin_dloc, counts)
        lg = layers[li]["gamma"].reshape(1, D)
        lb = layers[li]["beta"].reshape(1, D)
        if li < NL - 1:
            nxt = layers[li + 1]
            h, m0, m1, base = _t_mid(
                h, agg, lg, lb,
                nxt["node_W"][0], nxt["node_b"][0].reshape(1, D),
                nxt["node_W"][1], nxt["node_b"][1].reshape(1, D),
                s_mat, ew_mat(nxt))
        else:
            h = _t_out(h, agg, lg, lb)

    return h.reshape(1, N, D)
